# Initial kernel scaffold; baseline (speedup 1.0000x reference)
import jax
import jax.numpy as jnp
from jax.experimental import pallas as pl


def kernel(x, ln1_g, ln1_b, ln2_g, ln2_b, Wq, bq, Wk, bk, Wv, bv, Wo, bo, Wr, W1, b1, W2, b2):
    raise NotImplementedError("write your pallas kernel here")


# speedup vs baseline: 1.6669x; 1.6669x over previous
import functools

import jax
import jax.numpy as jnp
from jax import lax
from jax.experimental import pallas as pl
from jax.experimental.pallas import tpu as pltpu

EMBED_DIM = 768
NUM_HEADS = 12
NUM_EXPERTS = 8
TOPK = 2
LB_W = 0.01
FF = EMBED_DIM * 4

S_BLK = 256
Q_BLK = 512
M_BLK = 128
F_BLK = 768
N_TOK = 2048
N_ENTRY = N_TOK * TOPK
R_PAD = N_ENTRY + NUM_EXPERTS * M_BLK
NUM_M_BLKS = R_PAD // M_BLK
NUM_F_BLKS = FF // F_BLK


def _ln(x, g, b):
    m = jnp.mean(x, -1, keepdims=True)
    v = jnp.mean((x - m) * (x - m), -1, keepdims=True)
    return (x - m) * lax.rsqrt(v + 1e-5) * g + b


def _qkv_body(x_ref, g_ref, b_ref, wq_ref, bq_ref, wk_ref, bk_ref,
              wv_ref, bv_ref, q_out, k_out, v_out):
    h = _ln(x_ref[...], g_ref[...], b_ref[...])
    q_out[...] = jnp.dot(h, wq_ref[...], preferred_element_type=jnp.float32) + bq_ref[...]
    k_out[...] = jnp.dot(h, wk_ref[...], preferred_element_type=jnp.float32) + bk_ref[...]
    v_out[...] = jnp.dot(h, wv_ref[...], preferred_element_type=jnp.float32) + bv_ref[...]


def _qkv(x, ln1_g, ln1_b, Wq, bq, Wk, bk, Wv, bv):
    D = EMBED_DIM
    nblk = N_TOK // S_BLK
    row = pl.BlockSpec((S_BLK, D), lambda i: (i, 0))
    full = pl.BlockSpec((D, D), lambda i: (0, 0))
    vec = pl.BlockSpec((1, D), lambda i: (0, 0))
    out = jax.ShapeDtypeStruct((N_TOK, D), jnp.float32)
    return pl.pallas_call(
        _qkv_body,
        grid=(nblk,),
        in_specs=[row, vec, vec, full, vec, full, vec, full, vec],
        out_specs=[row, row, row],
        out_shape=[out, out, out],
    )(x, ln1_g, ln1_b, Wq, bq, Wk, bk, Wv, bv)


def _attn_body(q_ref, k_ref, v_ref, o_ref):
    dh = EMBED_DIM // NUM_HEADS
    s = lax.dot_general(q_ref[0], k_ref[0],
                        (((1,), (1,)), ((), ())),
                        preferred_element_type=jnp.float32)
    s = s * (1.0 / jnp.sqrt(jnp.float32(dh)))
    m = jnp.max(s, axis=-1, keepdims=True)
    p = jnp.exp(s - m)
    p = p / jnp.sum(p, axis=-1, keepdims=True)
    o_ref[0] = jnp.dot(p, v_ref[0], preferred_element_type=jnp.float32)


def _attention(q3, k3, v3):
    dh = EMBED_DIM // NUM_HEADS
    nq = N_TOK // Q_BLK
    qspec = pl.BlockSpec((1, Q_BLK, dh), lambda h, i: (h, i, 0))
    kvspec = pl.BlockSpec((1, N_TOK, dh), lambda h, i: (h, 0, 0))
    return pl.pallas_call(
        _attn_body,
        grid=(NUM_HEADS, nq),
        in_specs=[qspec, kvspec, kvspec],
        out_specs=qspec,
        out_shape=jax.ShapeDtypeStruct((NUM_HEADS, N_TOK, dh), jnp.float32),
    )(q3, k3, v3)


def _router_body(x_ref, o_ref, wo_ref, bo_ref, g_ref, b_ref, wr_ref,
                 x2_out, t_out, topi_out, gates_out, counts_out, psum_out):
    i = pl.program_id(0)
    E = NUM_EXPERTS
    x2 = x_ref[...] + jnp.dot(o_ref[...], wo_ref[...],
                              preferred_element_type=jnp.float32) + bo_ref[...]
    x2_out[...] = x2
    t = _ln(x2, g_ref[...], b_ref[...])
    t_out[...] = t
    logits = jnp.dot(t, wr_ref[...], preferred_element_type=jnp.float32)
    lm = jnp.max(logits, axis=-1, keepdims=True)
    pe = jnp.exp(logits - lm)
    probs = pe / jnp.sum(pe, axis=-1, keepdims=True)
    iota = lax.broadcasted_iota(jnp.int32, probs.shape, 1)
    m1 = jnp.max(probs, axis=-1, keepdims=True)
    i1 = jnp.min(jnp.where(probs == m1, iota, E), axis=-1, keepdims=True)
    probs2 = jnp.where(iota == i1, -1.0, probs)
    m2 = jnp.max(probs2, axis=-1, keepdims=True)
    i2 = jnp.min(jnp.where(probs2 == m2, iota, E), axis=-1, keepdims=True)
    denom = m1 + m2
    topi_out[...] = jnp.concatenate([i1, i2], axis=1)
    gates_out[...] = jnp.concatenate([m1 / denom, m2 / denom], axis=1)
    onehot = ((iota == i1) | (iota == i2)).astype(jnp.float32)
    cnt = jnp.sum(onehot, axis=0, keepdims=True)
    ps = jnp.sum(probs, axis=0, keepdims=True)

    @pl.when(i == 0)
    def _():
        counts_out[...] = jnp.zeros_like(counts_out)
        psum_out[...] = jnp.zeros_like(psum_out)

    counts_out[...] += cnt
    psum_out[...] += ps


def _router(x, o, Wo, bo, ln2_g, ln2_b, Wr):
    D, E = EMBED_DIM, NUM_EXPERTS
    nblk = N_TOK // S_BLK
    row = pl.BlockSpec((S_BLK, D), lambda i: (i, 0))
    full = pl.BlockSpec((D, D), lambda i: (0, 0))
    vec = pl.BlockSpec((1, D), lambda i: (0, 0))
    wr = pl.BlockSpec((D, E), lambda i: (0, 0))
    two = pl.BlockSpec((S_BLK, TOPK), lambda i: (i, 0))
    acc = pl.BlockSpec((1, E), lambda i: (0, 0))
    return pl.pallas_call(
        _router_body,
        grid=(nblk,),
        in_specs=[row, row, full, vec, vec, vec, wr],
        out_specs=[row, row, two, two, acc, acc],
        out_shape=[
            jax.ShapeDtypeStruct((N_TOK, D), jnp.float32),
            jax.ShapeDtypeStruct((N_TOK, D), jnp.float32),
            jax.ShapeDtypeStruct((N_TOK, TOPK), jnp.int32),
            jax.ShapeDtypeStruct((N_TOK, TOPK), jnp.float32),
            jax.ShapeDtypeStruct((1, E), jnp.float32),
            jax.ShapeDtypeStruct((1, E), jnp.float32),
        ],
    )(x, o, Wo, bo, ln2_g, ln2_b, Wr)


def _moe_body(be_ref, xs_ref, w1_ref, b1_ref, w2_ref, b2_ref, gate_ref, out_ref):
    j = pl.program_id(1)
    h = jnp.dot(xs_ref[...], w1_ref[0], preferred_element_type=jnp.float32)
    h = h + b1_ref[0]
    h = 0.5 * h * (1.0 + lax.erf(h * jnp.float32(0.7071067811865476)))
    contrib = jnp.dot(h, w2_ref[0], preferred_element_type=jnp.float32)

    @pl.when(j == 0)
    def _():
        out_ref[...] = jnp.zeros_like(out_ref)

    out_ref[...] += contrib

    @pl.when(j == NUM_F_BLKS - 1)
    def _():
        out_ref[...] = (out_ref[...] + b2_ref[0]) * gate_ref[...]


def _moe_grouped(block_expert, xs, W1, b1, W2, b2, row_gate):
    D, F = EMBED_DIM, FF
    grid_spec = pltpu.PrefetchScalarGridSpec(
        num_scalar_prefetch=1,
        grid=(NUM_M_BLKS, NUM_F_BLKS),
        in_specs=[
            pl.BlockSpec((M_BLK, D), lambda i, j, be: (i, 0)),
            pl.BlockSpec((1, D, F_BLK), lambda i, j, be: (be[i], 0, j)),
            pl.BlockSpec((1, 1, F_BLK), lambda i, j, be: (be[i], 0, j)),
            pl.BlockSpec((1, F_BLK, D), lambda i, j, be: (be[i], j, 0)),
            pl.BlockSpec((1, 1, D), lambda i, j, be: (be[i], 0, 0)),
            pl.BlockSpec((M_BLK, 1), lambda i, j, be: (i, 0)),
        ],
        out_specs=pl.BlockSpec((M_BLK, D), lambda i, j, be: (i, 0)),
    )
    return pl.pallas_call(
        _moe_body,
        grid_spec=grid_spec,
        out_shape=jax.ShapeDtypeStruct((R_PAD, D), jnp.float32),
    )(block_expert, xs, W1, b1.reshape(NUM_EXPERTS, 1, F),
      W2, b2.reshape(NUM_EXPERTS, 1, D), row_gate)


def _final_body(x2_ref, r0_ref, r1_ref, counts_ref, psum_ref, out_ref, lb_ref):
    i = pl.program_id(0)
    out_ref[...] = x2_ref[...] + r0_ref[...] + r1_ref[...]

    @pl.when(i == 0)
    def _():
        frac = counts_ref[...] / jnp.float32(N_TOK * TOPK)
        pmean = psum_ref[...] / jnp.float32(N_TOK)
        lb_ref[...] = (LB_W * NUM_EXPERTS) * jnp.sum(
            frac * pmean, keepdims=True).reshape(1, 1)


def _final(x2, r0, r1, counts, psum):
    D, E = EMBED_DIM, NUM_EXPERTS
    nblk = N_TOK // S_BLK
    row = pl.BlockSpec((S_BLK, D), lambda i: (i, 0))
    acc = pl.BlockSpec((1, E), lambda i: (0, 0))
    one = pl.BlockSpec((1, 1), lambda i: (0, 0))
    return pl.pallas_call(
        _final_body,
        grid=(nblk,),
        in_specs=[row, row, row, acc, acc],
        out_specs=[row, one],
        out_shape=[
            jax.ShapeDtypeStruct((N_TOK, D), jnp.float32),
            jax.ShapeDtypeStruct((1, 1), jnp.float32),
        ],
    )(x2, r0, r1, counts, psum)


def _routing_metadata(topi, gates, counts_f):
    flat_e = topi.reshape(-1)
    flat_g = gates.reshape(-1)
    flat_tok = (jnp.arange(N_ENTRY, dtype=jnp.int32) // TOPK)
    counts = counts_f.reshape(-1).astype(jnp.int32)
    padded = ((counts + M_BLK - 1) // M_BLK) * M_BLK
    start = jnp.concatenate([jnp.zeros((1,), jnp.int32),
                             jnp.cumsum(padded)[:-1].astype(jnp.int32)])
    cstart = jnp.concatenate([jnp.zeros((1,), jnp.int32),
                              jnp.cumsum(counts)[:-1].astype(jnp.int32)])
    order = jnp.argsort(flat_e, stable=True).astype(jnp.int32)
    sorted_e = flat_e[order]
    r = jnp.arange(N_ENTRY, dtype=jnp.int32)
    p_arr = start[sorted_e] + (r - cstart[sorted_e])
    pos_of_entry = jnp.zeros((N_ENTRY,), jnp.int32).at[order].set(p_arr)
    sorted_ids = jnp.zeros((R_PAD,), jnp.int32).at[p_arr].set(flat_tok[order])
    row_gate = jnp.zeros((R_PAD,), jnp.float32).at[p_arr].set(flat_g[order])
    ends = jnp.cumsum(padded).astype(jnp.int32)
    blk_base = jnp.arange(NUM_M_BLKS, dtype=jnp.int32) * M_BLK
    block_expert = jnp.clip(
        jnp.searchsorted(ends, blk_base, side="right").astype(jnp.int32),
        0, NUM_EXPERTS - 1)
    pos0 = pos_of_entry[0::TOPK]
    pos1 = pos_of_entry[1::TOPK]
    return sorted_ids, row_gate, block_expert, pos0, pos1


def kernel(x, ln1_g, ln1_b, ln2_g, ln2_b, Wq, bq, Wk, bk, Wv, bv,
           Wo, bo, Wr, W1, b1, W2, b2):
    B, S, D = x.shape
    x2d = x.reshape(S, D)
    v1 = lambda a: a.reshape(1, D)
    q, k, v = _qkv(x2d, v1(ln1_g), v1(ln1_b), Wq, v1(bq), Wk, v1(bk), Wv, v1(bv))
    H, dh = NUM_HEADS, D // NUM_HEADS
    to3 = lambda a: a.reshape(S, H, dh).transpose(1, 0, 2)
    o3 = _attention(to3(q), to3(k), to3(v))
    o = o3.transpose(1, 0, 2).reshape(S, D)
    x2, t, topi, gates, counts, psum = _router(
        x2d, o, Wo, v1(bo), v1(ln2_g), v1(ln2_b), Wr)
    sorted_ids, row_gate, block_expert, pos0, pos1 = _routing_metadata(
        topi, gates, counts)
    xs = t[sorted_ids]
    ys = _moe_grouped(block_expert, xs, W1, b1, W2, b2,
                      row_gate.reshape(R_PAD, 1))
    r0 = ys[pos0]
    r1 = ys[pos1]
    out, lb = _final(x2, r0, r1, counts, psum)
    return (out.reshape(B, S, D), lb.reshape(()))


# speedup vs baseline: 2.1524x; 1.2912x over previous
import functools

import jax
import jax.numpy as jnp
from jax import lax
from jax.experimental import pallas as pl
from jax.experimental.pallas import tpu as pltpu

EMBED_DIM = 768
NUM_HEADS = 12
NUM_EXPERTS = 8
TOPK = 2
LB_W = 0.01
FF = EMBED_DIM * 4

S_BLK = 256
Q_BLK = 512
M_BLK = 128
F_BLK = 768
N_TOK = 2048
N_ENTRY = N_TOK * TOPK
R_PAD = N_ENTRY + NUM_EXPERTS * M_BLK
NUM_M_BLKS = R_PAD // M_BLK
NUM_F_BLKS = FF // F_BLK


def _ln(x, g, b):
    m = jnp.mean(x, -1, keepdims=True)
    v = jnp.mean((x - m) * (x - m), -1, keepdims=True)
    return (x - m) * lax.rsqrt(v + 1e-5) * g + b


def _qkv_body(x_ref, g_ref, b_ref, wq_ref, bq_ref, wk_ref, bk_ref,
              wv_ref, bv_ref, q_out, k_out, v_out):
    h = _ln(x_ref[...], g_ref[...], b_ref[...])
    q_out[...] = jnp.dot(h, wq_ref[...], preferred_element_type=jnp.float32) + bq_ref[...]
    k_out[...] = jnp.dot(h, wk_ref[...], preferred_element_type=jnp.float32) + bk_ref[...]
    v_out[...] = jnp.dot(h, wv_ref[...], preferred_element_type=jnp.float32) + bv_ref[...]


def _qkv(x, ln1_g, ln1_b, Wq, bq, Wk, bk, Wv, bv):
    D = EMBED_DIM
    nblk = N_TOK // S_BLK
    row = pl.BlockSpec((S_BLK, D), lambda i: (i, 0))
    full = pl.BlockSpec((D, D), lambda i: (0, 0))
    vec = pl.BlockSpec((1, D), lambda i: (0, 0))
    out = jax.ShapeDtypeStruct((N_TOK, D), jnp.float32)
    return pl.pallas_call(
        _qkv_body,
        grid=(nblk,),
        in_specs=[row, vec, vec, full, vec, full, vec, full, vec],
        out_specs=[row, row, row],
        out_shape=[out, out, out],
    )(x, ln1_g, ln1_b, Wq, bq, Wk, bk, Wv, bv)


def _attn_body(q_ref, k_ref, v_ref, o_ref):
    dh = EMBED_DIM // NUM_HEADS
    s = lax.dot_general(q_ref[0], k_ref[0],
                        (((1,), (1,)), ((), ())),
                        preferred_element_type=jnp.float32)
    s = s * (1.0 / jnp.sqrt(jnp.float32(dh)))
    m = jnp.max(s, axis=-1, keepdims=True)
    p = jnp.exp(s - m)
    p = p / jnp.sum(p, axis=-1, keepdims=True)
    o_ref[0] = jnp.dot(p, v_ref[0], preferred_element_type=jnp.float32)


def _attention(q3, k3, v3):
    dh = EMBED_DIM // NUM_HEADS
    nq = N_TOK // Q_BLK
    qspec = pl.BlockSpec((1, Q_BLK, dh), lambda h, i: (h, i, 0))
    kvspec = pl.BlockSpec((1, N_TOK, dh), lambda h, i: (h, 0, 0))
    return pl.pallas_call(
        _attn_body,
        grid=(NUM_HEADS, nq),
        in_specs=[qspec, kvspec, kvspec],
        out_specs=qspec,
        out_shape=jax.ShapeDtypeStruct((NUM_HEADS, N_TOK, dh), jnp.float32),
    )(q3, k3, v3)


def _router_body(x_ref, o_ref, wo_ref, bo_ref, g_ref, b_ref, wr_ref,
                 x2_out, t_out, topi_out, gates_out, counts_out, psum_out):
    i = pl.program_id(0)
    E = NUM_EXPERTS
    x2 = x_ref[...] + jnp.dot(o_ref[...], wo_ref[...],
                              preferred_element_type=jnp.float32) + bo_ref[...]
    x2_out[...] = x2
    t = _ln(x2, g_ref[...], b_ref[...])
    t_out[...] = t
    logits = jnp.dot(t, wr_ref[...], preferred_element_type=jnp.float32)
    lm = jnp.max(logits, axis=-1, keepdims=True)
    pe = jnp.exp(logits - lm)
    probs = pe / jnp.sum(pe, axis=-1, keepdims=True)
    iota = lax.broadcasted_iota(jnp.int32, probs.shape, 1)
    m1 = jnp.max(probs, axis=-1, keepdims=True)
    i1 = jnp.min(jnp.where(probs == m1, iota, E), axis=-1, keepdims=True)
    probs2 = jnp.where(iota == i1, -1.0, probs)
    m2 = jnp.max(probs2, axis=-1, keepdims=True)
    i2 = jnp.min(jnp.where(probs2 == m2, iota, E), axis=-1, keepdims=True)
    denom = m1 + m2
    topi_out[...] = jnp.concatenate([i1, i2], axis=1)
    gates_out[...] = jnp.concatenate([m1 / denom, m2 / denom], axis=1)
    onehot = ((iota == i1) | (iota == i2)).astype(jnp.float32)
    cnt = jnp.sum(onehot, axis=0, keepdims=True)
    ps = jnp.sum(probs, axis=0, keepdims=True)

    @pl.when(i == 0)
    def _():
        counts_out[...] = jnp.zeros_like(counts_out)
        psum_out[...] = jnp.zeros_like(psum_out)

    counts_out[...] += cnt
    psum_out[...] += ps


def _router(x, o, Wo, bo, ln2_g, ln2_b, Wr):
    D, E = EMBED_DIM, NUM_EXPERTS
    nblk = N_TOK // S_BLK
    row = pl.BlockSpec((S_BLK, D), lambda i: (i, 0))
    full = pl.BlockSpec((D, D), lambda i: (0, 0))
    vec = pl.BlockSpec((1, D), lambda i: (0, 0))
    wr = pl.BlockSpec((D, E), lambda i: (0, 0))
    two = pl.BlockSpec((S_BLK, TOPK), lambda i: (i, 0))
    acc = pl.BlockSpec((1, E), lambda i: (0, 0))
    return pl.pallas_call(
        _router_body,
        grid=(nblk,),
        in_specs=[row, row, full, vec, vec, vec, wr],
        out_specs=[row, row, two, two, acc, acc],
        out_shape=[
            jax.ShapeDtypeStruct((N_TOK, D), jnp.float32),
            jax.ShapeDtypeStruct((N_TOK, D), jnp.float32),
            jax.ShapeDtypeStruct((N_TOK, TOPK), jnp.int32),
            jax.ShapeDtypeStruct((N_TOK, TOPK), jnp.float32),
            jax.ShapeDtypeStruct((1, E), jnp.float32),
            jax.ShapeDtypeStruct((1, E), jnp.float32),
        ],
    )(x, o, Wo, bo, ln2_g, ln2_b, Wr)


def _moe_body(be_ref, xs_ref, w1_ref, b1_ref, w2_ref, b2_ref, gate_ref, out_ref):
    h = jnp.dot(xs_ref[...], w1_ref[0], preferred_element_type=jnp.float32)
    h = h + b1_ref[0]
    h = 0.5 * h * (1.0 + lax.erf(h * jnp.float32(0.7071067811865476)))
    y = jnp.dot(h, w2_ref[0], preferred_element_type=jnp.float32)
    out_ref[...] = (y + b2_ref[0]) * gate_ref[...]


def _moe_grouped(block_expert, xs, W1, b1, W2, b2, row_gate):
    D, F = EMBED_DIM, FF
    grid_spec = pltpu.PrefetchScalarGridSpec(
        num_scalar_prefetch=1,
        grid=(NUM_M_BLKS,),
        in_specs=[
            pl.BlockSpec((M_BLK, D), lambda i, be: (i, 0)),
            pl.BlockSpec((1, D, F), lambda i, be: (be[i], 0, 0)),
            pl.BlockSpec((1, 1, F), lambda i, be: (be[i], 0, 0)),
            pl.BlockSpec((1, F, D), lambda i, be: (be[i], 0, 0)),
            pl.BlockSpec((1, 1, D), lambda i, be: (be[i], 0, 0)),
            pl.BlockSpec((M_BLK, 1), lambda i, be: (i, 0)),
        ],
        out_specs=pl.BlockSpec((M_BLK, D), lambda i, be: (i, 0)),
    )
    return pl.pallas_call(
        _moe_body,
        grid_spec=grid_spec,
        out_shape=jax.ShapeDtypeStruct((R_PAD, D), jnp.float32),
    )(block_expert, xs, W1, b1.reshape(NUM_EXPERTS, 1, F),
      W2, b2.reshape(NUM_EXPERTS, 1, D), row_gate)


def _final_body(x2_ref, r0_ref, r1_ref, counts_ref, psum_ref, out_ref, lb_ref):
    i = pl.program_id(0)
    out_ref[...] = x2_ref[...] + r0_ref[...] + r1_ref[...]

    @pl.when(i == 0)
    def _():
        frac = counts_ref[...] / jnp.float32(N_TOK * TOPK)
        pmean = psum_ref[...] / jnp.float32(N_TOK)
        lb_ref[...] = (LB_W * NUM_EXPERTS) * jnp.sum(
            frac * pmean, keepdims=True).reshape(1, 1)


def _final(x2, r0, r1, counts, psum):
    D, E = EMBED_DIM, NUM_EXPERTS
    nblk = N_TOK // S_BLK
    row = pl.BlockSpec((S_BLK, D), lambda i: (i, 0))
    acc = pl.BlockSpec((1, E), lambda i: (0, 0))
    one = pl.BlockSpec((1, 1), lambda i: (0, 0))
    return pl.pallas_call(
        _final_body,
        grid=(nblk,),
        in_specs=[row, row, row, acc, acc],
        out_specs=[row, one],
        out_shape=[
            jax.ShapeDtypeStruct((N_TOK, D), jnp.float32),
            jax.ShapeDtypeStruct((1, 1), jnp.float32),
        ],
    )(x2, r0, r1, counts, psum)


def _routing_metadata(topi, gates, counts_f):
    flat_e = topi.reshape(-1)
    flat_g = gates.reshape(-1)
    flat_tok = (jnp.arange(N_ENTRY, dtype=jnp.int32) // TOPK)
    counts = counts_f.reshape(-1).astype(jnp.int32)
    padded = ((counts + M_BLK - 1) // M_BLK) * M_BLK
    start = jnp.concatenate([jnp.zeros((1,), jnp.int32),
                             jnp.cumsum(padded)[:-1].astype(jnp.int32)])
    cstart = jnp.concatenate([jnp.zeros((1,), jnp.int32),
                              jnp.cumsum(counts)[:-1].astype(jnp.int32)])
    order = jnp.argsort(flat_e, stable=True).astype(jnp.int32)
    sorted_e = flat_e[order]
    r = jnp.arange(N_ENTRY, dtype=jnp.int32)
    p_arr = start[sorted_e] + (r - cstart[sorted_e])
    pos_of_entry = jnp.zeros((N_ENTRY,), jnp.int32).at[order].set(p_arr)
    sorted_ids = jnp.zeros((R_PAD,), jnp.int32).at[p_arr].set(flat_tok[order])
    row_gate = jnp.zeros((R_PAD,), jnp.float32).at[p_arr].set(flat_g[order])
    ends = jnp.cumsum(padded).astype(jnp.int32)
    blk_base = jnp.arange(NUM_M_BLKS, dtype=jnp.int32) * M_BLK
    block_expert = jnp.clip(
        jnp.searchsorted(ends, blk_base, side="right").astype(jnp.int32),
        0, NUM_EXPERTS - 1)
    pos0 = pos_of_entry[0::TOPK]
    pos1 = pos_of_entry[1::TOPK]
    return sorted_ids, row_gate, block_expert, pos0, pos1


def kernel(x, ln1_g, ln1_b, ln2_g, ln2_b, Wq, bq, Wk, bk, Wv, bv,
           Wo, bo, Wr, W1, b1, W2, b2):
    B, S, D = x.shape
    x2d = x.reshape(S, D)
    v1 = lambda a: a.reshape(1, D)
    q, k, v = _qkv(x2d, v1(ln1_g), v1(ln1_b), Wq, v1(bq), Wk, v1(bk), Wv, v1(bv))
    H, dh = NUM_HEADS, D // NUM_HEADS
    to3 = lambda a: a.reshape(S, H, dh).transpose(1, 0, 2)
    o3 = _attention(to3(q), to3(k), to3(v))
    o = o3.transpose(1, 0, 2).reshape(S, D)
    x2, t, topi, gates, counts, psum = _router(
        x2d, o, Wo, v1(bo), v1(ln2_g), v1(ln2_b), Wr)
    sorted_ids, row_gate, block_expert, pos0, pos1 = _routing_metadata(
        topi, gates, counts)
    xs = t[sorted_ids]
    ys = _moe_grouped(block_expert, xs, W1, b1, W2, b2,
                      row_gate.reshape(R_PAD, 1))
    r0 = ys[pos0]
    r1 = ys[pos1]
    out, lb = _final(x2, r0, r1, counts, psum)
    return (out.reshape(B, S, D), lb.reshape(()))


# speedup vs baseline: 2.2331x; 1.0375x over previous
import functools

import jax
import jax.numpy as jnp
from jax import lax
from jax.experimental import pallas as pl
from jax.experimental.pallas import tpu as pltpu
from jax.experimental.pallas import tpu_sc as plsc

EMBED_DIM = 768
NUM_HEADS = 12
NUM_EXPERTS = 8
TOPK = 2
LB_W = 0.01
FF = EMBED_DIM * 4

S_BLK = 256
Q_BLK = 512
M_BLK = 128
F_BLK = 768
N_TOK = 2048
N_ENTRY = N_TOK * TOPK
R_PAD = N_ENTRY + NUM_EXPERTS * M_BLK
NUM_M_BLKS = R_PAD // M_BLK
NUM_F_BLKS = FF // F_BLK

_SC_NC = 2
_SC_NS = 16
_SC_NW = _SC_NC * _SC_NS


def _sc_gather_rows(table, idx, nrows, ncols):
    b_per_w = nrows // _SC_NW
    mesh = plsc.VectorSubcoreMesh(core_axis_name="c", subcore_axis_name="s")

    @functools.partial(
        pl.kernel, mesh=mesh,
        out_type=jax.ShapeDtypeStruct((nrows, ncols), jnp.float32),
        scratch_types=[
            pltpu.VMEM((b_per_w,), jnp.int32),
            pltpu.VMEM((b_per_w, ncols), jnp.float32),
            pltpu.SemaphoreType.DMA,
        ],
    )
    def k(table_hbm, idx_hbm, out_hbm, idx_v, rows_v, sem):
        wid = lax.axis_index("s") * _SC_NC + lax.axis_index("c")
        base = wid * b_per_w
        pltpu.sync_copy(idx_hbm.at[pl.ds(base, b_per_w)], idx_v)
        pltpu.async_copy(table_hbm.at[idx_v], rows_v, sem).wait()
        pltpu.sync_copy(rows_v, out_hbm.at[pl.ds(base, b_per_w)])

    return k(table, idx)


def _ln(x, g, b):
    m = jnp.mean(x, -1, keepdims=True)
    v = jnp.mean((x - m) * (x - m), -1, keepdims=True)
    return (x - m) * lax.rsqrt(v + 1e-5) * g + b


def _qkv_body(x_ref, g_ref, b_ref, wq_ref, bq_ref, wk_ref, bk_ref,
              wv_ref, bv_ref, q_out, k_out, v_out):
    h = _ln(x_ref[...], g_ref[...], b_ref[...])
    q_out[...] = jnp.dot(h, wq_ref[...], preferred_element_type=jnp.float32) + bq_ref[...]
    k_out[...] = jnp.dot(h, wk_ref[...], preferred_element_type=jnp.float32) + bk_ref[...]
    v_out[...] = jnp.dot(h, wv_ref[...], preferred_element_type=jnp.float32) + bv_ref[...]


def _qkv(x, ln1_g, ln1_b, Wq, bq, Wk, bk, Wv, bv):
    D = EMBED_DIM
    nblk = N_TOK // S_BLK
    row = pl.BlockSpec((S_BLK, D), lambda i: (i, 0))
    full = pl.BlockSpec((D, D), lambda i: (0, 0))
    vec = pl.BlockSpec((1, D), lambda i: (0, 0))
    out = jax.ShapeDtypeStruct((N_TOK, D), jnp.float32)
    return pl.pallas_call(
        _qkv_body,
        grid=(nblk,),
        in_specs=[row, vec, vec, full, vec, full, vec, full, vec],
        out_specs=[row, row, row],
        out_shape=[out, out, out],
    )(x, ln1_g, ln1_b, Wq, bq, Wk, bk, Wv, bv)


def _attn_body(q_ref, k_ref, v_ref, o_ref):
    dh = EMBED_DIM // NUM_HEADS
    s = lax.dot_general(q_ref[0], k_ref[0],
                        (((1,), (1,)), ((), ())),
                        preferred_element_type=jnp.float32)
    s = s * (1.0 / jnp.sqrt(jnp.float32(dh)))
    m = jnp.max(s, axis=-1, keepdims=True)
    p = jnp.exp(s - m)
    p = p / jnp.sum(p, axis=-1, keepdims=True)
    o_ref[0] = jnp.dot(p, v_ref[0], preferred_element_type=jnp.float32)


def _attention(q3, k3, v3):
    dh = EMBED_DIM // NUM_HEADS
    nq = N_TOK // Q_BLK
    qspec = pl.BlockSpec((1, Q_BLK, dh), lambda h, i: (h, i, 0))
    kvspec = pl.BlockSpec((1, N_TOK, dh), lambda h, i: (h, 0, 0))
    return pl.pallas_call(
        _attn_body,
        grid=(NUM_HEADS, nq),
        in_specs=[qspec, kvspec, kvspec],
        out_specs=qspec,
        out_shape=jax.ShapeDtypeStruct((NUM_HEADS, N_TOK, dh), jnp.float32),
    )(q3, k3, v3)


def _router_body(x_ref, o_ref, wo_ref, bo_ref, g_ref, b_ref, wr_ref,
                 x2_out, t_out, topi_out, gates_out, counts_out, psum_out):
    i = pl.program_id(0)
    E = NUM_EXPERTS
    x2 = x_ref[...] + jnp.dot(o_ref[...], wo_ref[...],
                              preferred_element_type=jnp.float32) + bo_ref[...]
    x2_out[...] = x2
    t = _ln(x2, g_ref[...], b_ref[...])
    t_out[...] = t
    logits = jnp.dot(t, wr_ref[...], preferred_element_type=jnp.float32)
    lm = jnp.max(logits, axis=-1, keepdims=True)
    pe = jnp.exp(logits - lm)
    probs = pe / jnp.sum(pe, axis=-1, keepdims=True)
    iota = lax.broadcasted_iota(jnp.int32, probs.shape, 1)
    m1 = jnp.max(probs, axis=-1, keepdims=True)
    i1 = jnp.min(jnp.where(probs == m1, iota, E), axis=-1, keepdims=True)
    probs2 = jnp.where(iota == i1, -1.0, probs)
    m2 = jnp.max(probs2, axis=-1, keepdims=True)
    i2 = jnp.min(jnp.where(probs2 == m2, iota, E), axis=-1, keepdims=True)
    denom = m1 + m2
    topi_out[...] = jnp.concatenate([i1, i2], axis=1)
    gates_out[...] = jnp.concatenate([m1 / denom, m2 / denom], axis=1)
    onehot = ((iota == i1) | (iota == i2)).astype(jnp.float32)
    cnt = jnp.sum(onehot, axis=0, keepdims=True)
    ps = jnp.sum(probs, axis=0, keepdims=True)

    @pl.when(i == 0)
    def _():
        counts_out[...] = jnp.zeros_like(counts_out)
        psum_out[...] = jnp.zeros_like(psum_out)

    counts_out[...] += cnt
    psum_out[...] += ps


def _router(x, o, Wo, bo, ln2_g, ln2_b, Wr):
    D, E = EMBED_DIM, NUM_EXPERTS
    nblk = N_TOK // S_BLK
    row = pl.BlockSpec((S_BLK, D), lambda i: (i, 0))
    full = pl.BlockSpec((D, D), lambda i: (0, 0))
    vec = pl.BlockSpec((1, D), lambda i: (0, 0))
    wr = pl.BlockSpec((D, E), lambda i: (0, 0))
    two = pl.BlockSpec((S_BLK, TOPK), lambda i: (i, 0))
    acc = pl.BlockSpec((1, E), lambda i: (0, 0))
    return pl.pallas_call(
        _router_body,
        grid=(nblk,),
        in_specs=[row, row, full, vec, vec, vec, wr],
        out_specs=[row, row, two, two, acc, acc],
        out_shape=[
            jax.ShapeDtypeStruct((N_TOK, D), jnp.float32),
            jax.ShapeDtypeStruct((N_TOK, D), jnp.float32),
            jax.ShapeDtypeStruct((N_TOK, TOPK), jnp.int32),
            jax.ShapeDtypeStruct((N_TOK, TOPK), jnp.float32),
            jax.ShapeDtypeStruct((1, E), jnp.float32),
            jax.ShapeDtypeStruct((1, E), jnp.float32),
        ],
    )(x, o, Wo, bo, ln2_g, ln2_b, Wr)


def _moe_body(be_ref, xs_ref, w1_ref, b1_ref, w2_ref, b2_ref, gate_ref, out_ref):
    h = jnp.dot(xs_ref[...], w1_ref[0], preferred_element_type=jnp.float32)
    h = h + b1_ref[0]
    h = 0.5 * h * (1.0 + lax.erf(h * jnp.float32(0.7071067811865476)))
    y = jnp.dot(h, w2_ref[0], preferred_element_type=jnp.float32)
    out_ref[...] = (y + b2_ref[0]) * gate_ref[...]


def _moe_grouped(block_expert, xs, W1, b1, W2, b2, row_gate):
    D, F = EMBED_DIM, FF
    grid_spec = pltpu.PrefetchScalarGridSpec(
        num_scalar_prefetch=1,
        grid=(NUM_M_BLKS,),
        in_specs=[
            pl.BlockSpec((M_BLK, D), lambda i, be: (i, 0)),
            pl.BlockSpec((1, D, F), lambda i, be: (be[i], 0, 0)),
            pl.BlockSpec((1, 1, F), lambda i, be: (be[i], 0, 0)),
            pl.BlockSpec((1, F, D), lambda i, be: (be[i], 0, 0)),
            pl.BlockSpec((1, 1, D), lambda i, be: (be[i], 0, 0)),
            pl.BlockSpec((M_BLK, 1), lambda i, be: (i, 0)),
        ],
        out_specs=pl.BlockSpec((M_BLK, D), lambda i, be: (i, 0)),
    )
    return pl.pallas_call(
        _moe_body,
        grid_spec=grid_spec,
        out_shape=jax.ShapeDtypeStruct((R_PAD, D), jnp.float32),
    )(block_expert, xs, W1, b1.reshape(NUM_EXPERTS, 1, F),
      W2, b2.reshape(NUM_EXPERTS, 1, D), row_gate)


def _final_body(x2_ref, r0_ref, r1_ref, counts_ref, psum_ref, out_ref, lb_ref):
    i = pl.program_id(0)
    out_ref[...] = x2_ref[...] + r0_ref[...] + r1_ref[...]

    @pl.when(i == 0)
    def _():
        frac = counts_ref[...] / jnp.float32(N_TOK * TOPK)
        pmean = psum_ref[...] / jnp.float32(N_TOK)
        lb_ref[...] = (LB_W * NUM_EXPERTS) * jnp.sum(
            frac * pmean, keepdims=True).reshape(1, 1)


def _final(x2, rows, counts, psum):
    D, E = EMBED_DIM, NUM_EXPERTS
    nblk = N_TOK // S_BLK
    row = pl.BlockSpec((S_BLK, D), lambda i: (i, 0))
    row1 = pl.BlockSpec((S_BLK, D), lambda i: (i + nblk, 0))
    acc = pl.BlockSpec((1, E), lambda i: (0, 0))
    one = pl.BlockSpec((1, 1), lambda i: (0, 0))
    return pl.pallas_call(
        _final_body,
        grid=(nblk,),
        in_specs=[row, row, row1, acc, acc],
        out_specs=[row, one],
        out_shape=[
            jax.ShapeDtypeStruct((N_TOK, D), jnp.float32),
            jax.ShapeDtypeStruct((1, 1), jnp.float32),
        ],
    )(x2, rows, rows, counts, psum)


def _routing_metadata(topi, gates, counts_f):
    flat_e = topi.reshape(-1)
    flat_g = gates.reshape(-1)
    flat_tok = (jnp.arange(N_ENTRY, dtype=jnp.int32) // TOPK)
    counts = counts_f.reshape(-1).astype(jnp.int32)
    padded = ((counts + M_BLK - 1) // M_BLK) * M_BLK
    start = jnp.concatenate([jnp.zeros((1,), jnp.int32),
                             jnp.cumsum(padded)[:-1].astype(jnp.int32)])
    cstart = jnp.concatenate([jnp.zeros((1,), jnp.int32),
                              jnp.cumsum(counts)[:-1].astype(jnp.int32)])
    order = jnp.argsort(flat_e, stable=True).astype(jnp.int32)
    sorted_e = flat_e[order]
    r = jnp.arange(N_ENTRY, dtype=jnp.int32)
    p_arr = start[sorted_e] + (r - cstart[sorted_e])
    pos_of_entry = jnp.zeros((N_ENTRY,), jnp.int32).at[order].set(p_arr)
    sorted_ids = jnp.zeros((R_PAD,), jnp.int32).at[p_arr].set(flat_tok[order])
    row_gate = jnp.zeros((R_PAD,), jnp.float32).at[p_arr].set(flat_g[order])
    ends = jnp.cumsum(padded).astype(jnp.int32)
    blk_base = jnp.arange(NUM_M_BLKS, dtype=jnp.int32) * M_BLK
    block_expert = jnp.clip(
        jnp.searchsorted(ends, blk_base, side="right").astype(jnp.int32),
        0, NUM_EXPERTS - 1)
    pos0 = pos_of_entry[0::TOPK]
    pos1 = pos_of_entry[1::TOPK]
    return sorted_ids, row_gate, block_expert, pos0, pos1


def kernel(x, ln1_g, ln1_b, ln2_g, ln2_b, Wq, bq, Wk, bk, Wv, bv,
           Wo, bo, Wr, W1, b1, W2, b2):
    B, S, D = x.shape
    x2d = x.reshape(S, D)
    v1 = lambda a: a.reshape(1, D)
    q, k, v = _qkv(x2d, v1(ln1_g), v1(ln1_b), Wq, v1(bq), Wk, v1(bk), Wv, v1(bv))
    H, dh = NUM_HEADS, D // NUM_HEADS
    to3 = lambda a: a.reshape(S, H, dh).transpose(1, 0, 2)
    o3 = _attention(to3(q), to3(k), to3(v))
    o = o3.transpose(1, 0, 2).reshape(S, D)
    x2, t, topi, gates, counts, psum = _router(
        x2d, o, Wo, v1(bo), v1(ln2_g), v1(ln2_b), Wr)
    sorted_ids, row_gate, block_expert, pos0, pos1 = _routing_metadata(
        topi, gates, counts)
    xs = _sc_gather_rows(t, sorted_ids, R_PAD, D)
    ys = _moe_grouped(block_expert, xs, W1, b1, W2, b2,
                      row_gate.reshape(R_PAD, 1))
    poscat = jnp.concatenate([pos0, pos1])
    rows = _sc_gather_rows(ys, poscat, N_ENTRY, D)
    out, lb = _final(x2, rows, counts, psum)
    return (out.reshape(B, S, D), lb.reshape(()))


# speedup vs baseline: 2.2528x; 1.0088x over previous
import functools

import jax
import jax.numpy as jnp
from jax import lax
from jax.experimental import pallas as pl
from jax.experimental.pallas import tpu as pltpu
from jax.experimental.pallas import tpu_sc as plsc

EMBED_DIM = 768
NUM_HEADS = 12
NUM_EXPERTS = 8
TOPK = 2
LB_W = 0.01
FF = EMBED_DIM * 4

S_BLK = 256
Q_BLK = 512
M_BLK = 128
F_BLK = 768
N_TOK = 2048
N_ENTRY = N_TOK * TOPK
R_PAD = N_ENTRY + NUM_EXPERTS * M_BLK
NUM_M_BLKS = R_PAD // M_BLK
NUM_F_BLKS = FF // F_BLK

_SC_NC = 2
_SC_NS = 16
_SC_NW = _SC_NC * _SC_NS


def _sc_gather_rows(table, idx, nrows, ncols):
    b_per_w = nrows // _SC_NW
    mesh = plsc.VectorSubcoreMesh(core_axis_name="c", subcore_axis_name="s")

    @functools.partial(
        pl.kernel, mesh=mesh,
        out_type=jax.ShapeDtypeStruct((nrows, ncols), jnp.float32),
        scratch_types=[
            pltpu.VMEM((b_per_w,), jnp.int32),
            pltpu.VMEM((b_per_w, ncols), jnp.float32),
            pltpu.SemaphoreType.DMA,
        ],
    )
    def k(table_hbm, idx_hbm, out_hbm, idx_v, rows_v, sem):
        wid = lax.axis_index("s") * _SC_NC + lax.axis_index("c")
        base = wid * b_per_w
        pltpu.sync_copy(idx_hbm.at[pl.ds(base, b_per_w)], idx_v)
        pltpu.async_copy(table_hbm.at[idx_v], rows_v, sem).wait()
        pltpu.sync_copy(rows_v, out_hbm.at[pl.ds(base, b_per_w)])

    return k(table, idx)


def _ln(x, g, b):
    m = jnp.mean(x, -1, keepdims=True)
    v = jnp.mean((x - m) * (x - m), -1, keepdims=True)
    return (x - m) * lax.rsqrt(v + 1e-5) * g + b


def _qkv_body(x_ref, g_ref, b_ref, wq_ref, bq_ref, wk_ref, bk_ref,
              wv_ref, bv_ref, q_out, k_out, v_out):
    h = _ln(x_ref[...], g_ref[...], b_ref[...])
    q_out[...] = jnp.dot(h, wq_ref[...], preferred_element_type=jnp.float32) + bq_ref[...]
    k_out[...] = jnp.dot(h, wk_ref[...], preferred_element_type=jnp.float32) + bk_ref[...]
    v_out[...] = jnp.dot(h, wv_ref[...], preferred_element_type=jnp.float32) + bv_ref[...]


def _qkv(x, ln1_g, ln1_b, Wq, bq, Wk, bk, Wv, bv):
    D = EMBED_DIM
    nblk = N_TOK // S_BLK
    row = pl.BlockSpec((S_BLK, D), lambda i: (i, 0))
    full = pl.BlockSpec((D, D), lambda i: (0, 0))
    vec = pl.BlockSpec((1, D), lambda i: (0, 0))
    out = jax.ShapeDtypeStruct((N_TOK, D), jnp.float32)
    return pl.pallas_call(
        _qkv_body,
        grid=(nblk,),
        in_specs=[row, vec, vec, full, vec, full, vec, full, vec],
        out_specs=[row, row, row],
        out_shape=[out, out, out],
    )(x, ln1_g, ln1_b, Wq, bq, Wk, bk, Wv, bv)


def _attn_body(q_ref, k_ref, v_ref, o_ref):
    dh = EMBED_DIM // NUM_HEADS
    s = lax.dot_general(q_ref[0], k_ref[0],
                        (((1,), (1,)), ((), ())),
                        preferred_element_type=jnp.float32)
    s = s * (1.0 / jnp.sqrt(jnp.float32(dh)))
    m = jnp.max(s, axis=-1, keepdims=True)
    p = jnp.exp(s - m)
    o = jnp.dot(p, v_ref[0], preferred_element_type=jnp.float32)
    o_ref[0] = o * (1.0 / jnp.sum(p, axis=-1, keepdims=True))


def _attention(q3, k3, v3):
    dh = EMBED_DIM // NUM_HEADS
    nq = N_TOK // Q_BLK
    qspec = pl.BlockSpec((1, Q_BLK, dh), lambda h, i: (h, i, 0))
    kvspec = pl.BlockSpec((1, N_TOK, dh), lambda h, i: (h, 0, 0))
    return pl.pallas_call(
        _attn_body,
        grid=(NUM_HEADS, nq),
        in_specs=[qspec, kvspec, kvspec],
        out_specs=qspec,
        out_shape=jax.ShapeDtypeStruct((NUM_HEADS, N_TOK, dh), jnp.float32),
    )(q3, k3, v3)


def _router_body(x_ref, o_ref, wo_ref, bo_ref, g_ref, b_ref, wr_ref,
                 x2_out, t_out, topi_out, gates_out, counts_out, psum_out):
    i = pl.program_id(0)
    E = NUM_EXPERTS
    x2 = x_ref[...] + jnp.dot(o_ref[...], wo_ref[...],
                              preferred_element_type=jnp.float32) + bo_ref[...]
    x2_out[...] = x2
    t = _ln(x2, g_ref[...], b_ref[...])
    t_out[...] = t
    logits = jnp.dot(t, wr_ref[...], preferred_element_type=jnp.float32)
    lm = jnp.max(logits, axis=-1, keepdims=True)
    pe = jnp.exp(logits - lm)
    probs = pe / jnp.sum(pe, axis=-1, keepdims=True)
    iota = lax.broadcasted_iota(jnp.int32, probs.shape, 1)
    m1 = jnp.max(probs, axis=-1, keepdims=True)
    i1 = jnp.min(jnp.where(probs == m1, iota, E), axis=-1, keepdims=True)
    probs2 = jnp.where(iota == i1, -1.0, probs)
    m2 = jnp.max(probs2, axis=-1, keepdims=True)
    i2 = jnp.min(jnp.where(probs2 == m2, iota, E), axis=-1, keepdims=True)
    denom = m1 + m2
    topi_out[...] = jnp.concatenate([i1, i2], axis=1)
    gates_out[...] = jnp.concatenate([m1 / denom, m2 / denom], axis=1)
    onehot = ((iota == i1) | (iota == i2)).astype(jnp.float32)
    cnt = jnp.sum(onehot, axis=0, keepdims=True)
    ps = jnp.sum(probs, axis=0, keepdims=True)

    @pl.when(i == 0)
    def _():
        counts_out[...] = jnp.zeros_like(counts_out)
        psum_out[...] = jnp.zeros_like(psum_out)

    counts_out[...] += cnt
    psum_out[...] += ps


def _router(x, o, Wo, bo, ln2_g, ln2_b, Wr):
    D, E = EMBED_DIM, NUM_EXPERTS
    nblk = N_TOK // S_BLK
    row = pl.BlockSpec((S_BLK, D), lambda i: (i, 0))
    full = pl.BlockSpec((D, D), lambda i: (0, 0))
    vec = pl.BlockSpec((1, D), lambda i: (0, 0))
    wr = pl.BlockSpec((D, E), lambda i: (0, 0))
    two = pl.BlockSpec((S_BLK, TOPK), lambda i: (i, 0))
    acc = pl.BlockSpec((1, E), lambda i: (0, 0))
    return pl.pallas_call(
        _router_body,
        grid=(nblk,),
        in_specs=[row, row, full, vec, vec, vec, wr],
        out_specs=[row, row, two, two, acc, acc],
        out_shape=[
            jax.ShapeDtypeStruct((N_TOK, D), jnp.float32),
            jax.ShapeDtypeStruct((N_TOK, D), jnp.float32),
            jax.ShapeDtypeStruct((N_TOK, TOPK), jnp.int32),
            jax.ShapeDtypeStruct((N_TOK, TOPK), jnp.float32),
            jax.ShapeDtypeStruct((1, E), jnp.float32),
            jax.ShapeDtypeStruct((1, E), jnp.float32),
        ],
    )(x, o, Wo, bo, ln2_g, ln2_b, Wr)


def _moe_body(be_ref, xs_ref, w1_ref, b1_ref, w2_ref, b2_ref, gate_ref, out_ref):
    h = jnp.dot(xs_ref[...], w1_ref[0], preferred_element_type=jnp.float32)
    h = h + b1_ref[0]
    h = 0.5 * h * (1.0 + lax.erf(h * jnp.float32(0.7071067811865476)))
    y = jnp.dot(h, w2_ref[0], preferred_element_type=jnp.float32)
    out_ref[...] = (y + b2_ref[0]) * gate_ref[...]


def _moe_grouped(block_expert, xs, W1, b1, W2, b2, row_gate):
    D, F = EMBED_DIM, FF
    grid_spec = pltpu.PrefetchScalarGridSpec(
        num_scalar_prefetch=1,
        grid=(NUM_M_BLKS,),
        in_specs=[
            pl.BlockSpec((M_BLK, D), lambda i, be: (i, 0)),
            pl.BlockSpec((1, D, F), lambda i, be: (be[i], 0, 0)),
            pl.BlockSpec((1, 1, F), lambda i, be: (be[i], 0, 0)),
            pl.BlockSpec((1, F, D), lambda i, be: (be[i], 0, 0)),
            pl.BlockSpec((1, 1, D), lambda i, be: (be[i], 0, 0)),
            pl.BlockSpec((M_BLK, 1), lambda i, be: (i, 0)),
        ],
        out_specs=pl.BlockSpec((M_BLK, D), lambda i, be: (i, 0)),
    )
    return pl.pallas_call(
        _moe_body,
        grid_spec=grid_spec,
        out_shape=jax.ShapeDtypeStruct((R_PAD, D), jnp.float32),
    )(block_expert, xs, W1, b1.reshape(NUM_EXPERTS, 1, F),
      W2, b2.reshape(NUM_EXPERTS, 1, D), row_gate)


def _final_body(x2_ref, r0_ref, r1_ref, counts_ref, psum_ref, out_ref, lb_ref):
    i = pl.program_id(0)
    out_ref[...] = x2_ref[...] + r0_ref[...] + r1_ref[...]

    @pl.when(i == 0)
    def _():
        frac = counts_ref[...] / jnp.float32(N_TOK * TOPK)
        pmean = psum_ref[...] / jnp.float32(N_TOK)
        lb_ref[...] = (LB_W * NUM_EXPERTS) * jnp.sum(
            frac * pmean, keepdims=True).reshape(1, 1)


def _final(x2, rows, counts, psum):
    D, E = EMBED_DIM, NUM_EXPERTS
    nblk = N_TOK // S_BLK
    row = pl.BlockSpec((S_BLK, D), lambda i: (i, 0))
    row1 = pl.BlockSpec((S_BLK, D), lambda i: (i + nblk, 0))
    acc = pl.BlockSpec((1, E), lambda i: (0, 0))
    one = pl.BlockSpec((1, 1), lambda i: (0, 0))
    return pl.pallas_call(
        _final_body,
        grid=(nblk,),
        in_specs=[row, row, row1, acc, acc],
        out_specs=[row, one],
        out_shape=[
            jax.ShapeDtypeStruct((N_TOK, D), jnp.float32),
            jax.ShapeDtypeStruct((1, 1), jnp.float32),
        ],
    )(x2, rows, rows, counts, psum)


def _routing_metadata(topi, gates, counts_f):
    flat_e = topi.reshape(-1)
    flat_g = gates.reshape(-1)
    flat_tok = (jnp.arange(N_ENTRY, dtype=jnp.int32) // TOPK)
    counts = counts_f.reshape(-1).astype(jnp.int32)
    padded = ((counts + M_BLK - 1) // M_BLK) * M_BLK
    start = jnp.concatenate([jnp.zeros((1,), jnp.int32),
                             jnp.cumsum(padded)[:-1].astype(jnp.int32)])
    cstart = jnp.concatenate([jnp.zeros((1,), jnp.int32),
                              jnp.cumsum(counts)[:-1].astype(jnp.int32)])
    order = jnp.argsort(flat_e, stable=True).astype(jnp.int32)
    sorted_e = flat_e[order]
    r = jnp.arange(N_ENTRY, dtype=jnp.int32)
    p_arr = start[sorted_e] + (r - cstart[sorted_e])
    pos_of_entry = jnp.zeros((N_ENTRY,), jnp.int32).at[order].set(p_arr)
    sorted_ids = jnp.zeros((R_PAD,), jnp.int32).at[p_arr].set(flat_tok[order])
    row_gate = jnp.zeros((R_PAD,), jnp.float32).at[p_arr].set(flat_g[order])
    ends = jnp.cumsum(padded).astype(jnp.int32)
    blk_base = jnp.arange(NUM_M_BLKS, dtype=jnp.int32) * M_BLK
    block_expert = jnp.clip(
        jnp.searchsorted(ends, blk_base, side="right").astype(jnp.int32),
        0, NUM_EXPERTS - 1)
    pos0 = pos_of_entry[0::TOPK]
    pos1 = pos_of_entry[1::TOPK]
    return sorted_ids, row_gate, block_expert, pos0, pos1


def kernel(x, ln1_g, ln1_b, ln2_g, ln2_b, Wq, bq, Wk, bk, Wv, bv,
           Wo, bo, Wr, W1, b1, W2, b2):
    B, S, D = x.shape
    x2d = x.reshape(S, D)
    v1 = lambda a: a.reshape(1, D)
    q, k, v = _qkv(x2d, v1(ln1_g), v1(ln1_b), Wq, v1(bq), Wk, v1(bk), Wv, v1(bv))
    H, dh = NUM_HEADS, D // NUM_HEADS
    to3 = lambda a: a.reshape(S, H, dh).transpose(1, 0, 2)
    o3 = _attention(to3(q), to3(k), to3(v))
    o = o3.transpose(1, 0, 2).reshape(S, D)
    x2, t, topi, gates, counts, psum = _router(
        x2d, o, Wo, v1(bo), v1(ln2_g), v1(ln2_b), Wr)
    sorted_ids, row_gate, block_expert, pos0, pos1 = _routing_metadata(
        topi, gates, counts)
    xs = _sc_gather_rows(t, sorted_ids, R_PAD, D)
    ys = _moe_grouped(block_expert, xs, W1, b1, W2, b2,
                      row_gate.reshape(R_PAD, 1))
    poscat = jnp.concatenate([pos0, pos1])
    rows = _sc_gather_rows(ys, poscat, N_ENTRY, D)
    out, lb = _final(x2, rows, counts, psum)
    return (out.reshape(B, S, D), lb.reshape(()))


# speedup vs baseline: 2.2992x; 1.0206x over previous
import functools

import jax
import jax.numpy as jnp
from jax import lax
from jax.experimental import pallas as pl
from jax.experimental.pallas import tpu as pltpu
from jax.experimental.pallas import tpu_sc as plsc

EMBED_DIM = 768
NUM_HEADS = 12
NUM_EXPERTS = 8
TOPK = 2
LB_W = 0.01
FF = EMBED_DIM * 4

S_BLK = 256
Q_BLK = 1024
M_BLK = 128
F_BLK = 768
N_TOK = 2048
N_ENTRY = N_TOK * TOPK
R_PAD = N_ENTRY + NUM_EXPERTS * M_BLK
NUM_M_BLKS = R_PAD // M_BLK
NUM_F_BLKS = FF // F_BLK

_SC_NC = 2
_SC_NS = 16
_SC_NW = _SC_NC * _SC_NS


def _sc_gather_rows(table, idx, nrows, ncols):
    b_per_w = nrows // _SC_NW
    mesh = plsc.VectorSubcoreMesh(core_axis_name="c", subcore_axis_name="s")

    @functools.partial(
        pl.kernel, mesh=mesh,
        out_type=jax.ShapeDtypeStruct((nrows, ncols), jnp.float32),
        compiler_params=pltpu.CompilerParams(use_tc_tiling_on_sc=True),
        scratch_types=[
            pltpu.VMEM((b_per_w,), jnp.int32),
            pltpu.VMEM((b_per_w, ncols), jnp.float32),
            pltpu.SemaphoreType.DMA,
        ],
    )
    def k(table_hbm, idx_hbm, out_hbm, idx_v, rows_v, sem):
        wid = lax.axis_index("s") * _SC_NC + lax.axis_index("c")
        base = wid * b_per_w
        pltpu.sync_copy(idx_hbm.at[pl.ds(base, b_per_w)], idx_v)
        pltpu.async_copy(table_hbm.at[idx_v], rows_v, sem).wait()
        pltpu.sync_copy(rows_v, out_hbm.at[pl.ds(base, b_per_w)])

    return k(table, idx)


def _ln(x, g, b):
    m = jnp.mean(x, -1, keepdims=True)
    v = jnp.mean((x - m) * (x - m), -1, keepdims=True)
    return (x - m) * lax.rsqrt(v + 1e-5) * g + b


def _qkv_body(x_ref, g_ref, b_ref, wq_ref, bq_ref, wk_ref, bk_ref,
              wv_ref, bv_ref, q_out, k_out, v_out):
    h = _ln(x_ref[...], g_ref[...], b_ref[...])
    q_out[...] = jnp.dot(h, wq_ref[...], preferred_element_type=jnp.float32) + bq_ref[...]
    k_out[...] = jnp.dot(h, wk_ref[...], preferred_element_type=jnp.float32) + bk_ref[...]
    v_out[...] = jnp.dot(h, wv_ref[...], preferred_element_type=jnp.float32) + bv_ref[...]


def _qkv(x, ln1_g, ln1_b, Wq, bq, Wk, bk, Wv, bv):
    D = EMBED_DIM
    nblk = N_TOK // S_BLK
    row = pl.BlockSpec((S_BLK, D), lambda i: (i, 0))
    full = pl.BlockSpec((D, D), lambda i: (0, 0))
    vec = pl.BlockSpec((1, D), lambda i: (0, 0))
    out = jax.ShapeDtypeStruct((N_TOK, D), jnp.float32)
    return pl.pallas_call(
        _qkv_body,
        grid=(nblk,),
        in_specs=[row, vec, vec, full, vec, full, vec, full, vec],
        out_specs=[row, row, row],
        out_shape=[out, out, out],
    )(x, ln1_g, ln1_b, Wq, bq, Wk, bk, Wv, bv)


def _attn_body(q_ref, k_ref, v_ref, o_ref):
    dh = EMBED_DIM // NUM_HEADS
    s = lax.dot_general(q_ref[0], k_ref[0],
                        (((1,), (1,)), ((), ())),
                        preferred_element_type=jnp.float32)
    s = s * (1.0 / jnp.sqrt(jnp.float32(dh)))
    m = jnp.max(s, axis=-1, keepdims=True)
    p = jnp.exp(s - m)
    o = jnp.dot(p, v_ref[0], preferred_element_type=jnp.float32)
    o_ref[0] = o * (1.0 / jnp.sum(p, axis=-1, keepdims=True))


def _attention(q3, k3, v3):
    dh = EMBED_DIM // NUM_HEADS
    nq = N_TOK // Q_BLK
    qspec = pl.BlockSpec((1, Q_BLK, dh), lambda h, i: (h, i, 0))
    kvspec = pl.BlockSpec((1, N_TOK, dh), lambda h, i: (h, 0, 0))
    return pl.pallas_call(
        _attn_body,
        grid=(NUM_HEADS, nq),
        in_specs=[qspec, kvspec, kvspec],
        out_specs=qspec,
        out_shape=jax.ShapeDtypeStruct((NUM_HEADS, N_TOK, dh), jnp.float32),
    )(q3, k3, v3)


def _router_body(x_ref, o_ref, wo_ref, bo_ref, g_ref, b_ref, wr_ref,
                 x2_out, t_out, topi_out, gates_out, counts_out, psum_out):
    i = pl.program_id(0)
    E = NUM_EXPERTS
    x2 = x_ref[...] + jnp.dot(o_ref[...], wo_ref[...],
                              preferred_element_type=jnp.float32) + bo_ref[...]
    x2_out[...] = x2
    t = _ln(x2, g_ref[...], b_ref[...])
    t_out[...] = t
    logits = jnp.dot(t, wr_ref[...], preferred_element_type=jnp.float32)
    lm = jnp.max(logits, axis=-1, keepdims=True)
    pe = jnp.exp(logits - lm)
    probs = pe / jnp.sum(pe, axis=-1, keepdims=True)
    iota = lax.broadcasted_iota(jnp.int32, probs.shape, 1)
    m1 = jnp.max(probs, axis=-1, keepdims=True)
    i1 = jnp.min(jnp.where(probs == m1, iota, E), axis=-1, keepdims=True)
    probs2 = jnp.where(iota == i1, -1.0, probs)
    m2 = jnp.max(probs2, axis=-1, keepdims=True)
    i2 = jnp.min(jnp.where(probs2 == m2, iota, E), axis=-1, keepdims=True)
    denom = m1 + m2
    topi_out[...] = jnp.concatenate([i1, i2], axis=1)
    gates_out[...] = jnp.concatenate([m1 / denom, m2 / denom], axis=1)
    onehot = ((iota == i1) | (iota == i2)).astype(jnp.float32)
    cnt = jnp.sum(onehot, axis=0, keepdims=True)
    ps = jnp.sum(probs, axis=0, keepdims=True)

    @pl.when(i == 0)
    def _():
        counts_out[...] = jnp.zeros_like(counts_out)
        psum_out[...] = jnp.zeros_like(psum_out)

    counts_out[...] += cnt
    psum_out[...] += ps


def _router(x, o, Wo, bo, ln2_g, ln2_b, Wr):
    D, E = EMBED_DIM, NUM_EXPERTS
    nblk = N_TOK // S_BLK
    row = pl.BlockSpec((S_BLK, D), lambda i: (i, 0))
    full = pl.BlockSpec((D, D), lambda i: (0, 0))
    vec = pl.BlockSpec((1, D), lambda i: (0, 0))
    wr = pl.BlockSpec((D, E), lambda i: (0, 0))
    two = pl.BlockSpec((S_BLK, TOPK), lambda i: (i, 0))
    acc = pl.BlockSpec((1, E), lambda i: (0, 0))
    return pl.pallas_call(
        _router_body,
        grid=(nblk,),
        in_specs=[row, row, full, vec, vec, vec, wr],
        out_specs=[row, row, two, two, acc, acc],
        out_shape=[
            jax.ShapeDtypeStruct((N_TOK, D), jnp.float32),
            jax.ShapeDtypeStruct((N_TOK, D), jnp.float32),
            jax.ShapeDtypeStruct((N_TOK, TOPK), jnp.int32),
            jax.ShapeDtypeStruct((N_TOK, TOPK), jnp.float32),
            jax.ShapeDtypeStruct((1, E), jnp.float32),
            jax.ShapeDtypeStruct((1, E), jnp.float32),
        ],
    )(x, o, Wo, bo, ln2_g, ln2_b, Wr)


def _moe_body(be_ref, xs_ref, w1_ref, b1_ref, w2_ref, b2_ref, gate_ref, out_ref):
    h = jnp.dot(xs_ref[...], w1_ref[0], preferred_element_type=jnp.float32)
    h = h + b1_ref[0]
    h = 0.5 * h * (1.0 + lax.erf(h * jnp.float32(0.7071067811865476)))
    y = jnp.dot(h, w2_ref[0], preferred_element_type=jnp.float32)
    out_ref[...] = (y + b2_ref[0]) * gate_ref[...]


def _moe_grouped(block_expert, xs, W1, b1, W2, b2, row_gate):
    D, F = EMBED_DIM, FF
    grid_spec = pltpu.PrefetchScalarGridSpec(
        num_scalar_prefetch=1,
        grid=(NUM_M_BLKS,),
        in_specs=[
            pl.BlockSpec((M_BLK, D), lambda i, be: (i, 0)),
            pl.BlockSpec((1, D, F), lambda i, be: (be[i], 0, 0)),
            pl.BlockSpec((1, 1, F), lambda i, be: (be[i], 0, 0)),
            pl.BlockSpec((1, F, D), lambda i, be: (be[i], 0, 0)),
            pl.BlockSpec((1, 1, D), lambda i, be: (be[i], 0, 0)),
            pl.BlockSpec((M_BLK, 1), lambda i, be: (i, 0)),
        ],
        out_specs=pl.BlockSpec((M_BLK, D), lambda i, be: (i, 0)),
    )
    return pl.pallas_call(
        _moe_body,
        grid_spec=grid_spec,
        out_shape=jax.ShapeDtypeStruct((R_PAD, D), jnp.float32),
    )(block_expert, xs, W1, b1.reshape(NUM_EXPERTS, 1, F),
      W2, b2.reshape(NUM_EXPERTS, 1, D), row_gate)


def _final_body(x2_ref, r0_ref, r1_ref, counts_ref, psum_ref, out_ref, lb_ref):
    i = pl.program_id(0)
    out_ref[...] = x2_ref[...] + r0_ref[...] + r1_ref[...]

    @pl.when(i == 0)
    def _():
        frac = counts_ref[...] / jnp.float32(N_TOK * TOPK)
        pmean = psum_ref[...] / jnp.float32(N_TOK)
        lb_ref[...] = (LB_W * NUM_EXPERTS) * jnp.sum(
            frac * pmean, keepdims=True).reshape(1, 1)


def _final(x2, rows, counts, psum):
    D, E = EMBED_DIM, NUM_EXPERTS
    nblk = N_TOK // S_BLK
    row = pl.BlockSpec((S_BLK, D), lambda i: (i, 0))
    row1 = pl.BlockSpec((S_BLK, D), lambda i: (i + nblk, 0))
    acc = pl.BlockSpec((1, E), lambda i: (0, 0))
    one = pl.BlockSpec((1, 1), lambda i: (0, 0))
    return pl.pallas_call(
        _final_body,
        grid=(nblk,),
        in_specs=[row, row, row1, acc, acc],
        out_specs=[row, one],
        out_shape=[
            jax.ShapeDtypeStruct((N_TOK, D), jnp.float32),
            jax.ShapeDtypeStruct((1, 1), jnp.float32),
        ],
    )(x2, rows, rows, counts, psum)


def _routing_metadata(topi, gates, counts_f):
    flat_e = topi.reshape(-1)
    flat_g = gates.reshape(-1)
    flat_tok = (jnp.arange(N_ENTRY, dtype=jnp.int32) // TOPK)
    counts = counts_f.reshape(-1).astype(jnp.int32)
    padded = ((counts + M_BLK - 1) // M_BLK) * M_BLK
    start = jnp.concatenate([jnp.zeros((1,), jnp.int32),
                             jnp.cumsum(padded)[:-1].astype(jnp.int32)])
    cstart = jnp.concatenate([jnp.zeros((1,), jnp.int32),
                              jnp.cumsum(counts)[:-1].astype(jnp.int32)])
    order = jnp.argsort(flat_e, stable=True).astype(jnp.int32)
    sorted_e = flat_e[order]
    r = jnp.arange(N_ENTRY, dtype=jnp.int32)
    p_arr = start[sorted_e] + (r - cstart[sorted_e])
    pos_of_entry = jnp.zeros((N_ENTRY,), jnp.int32).at[order].set(p_arr)
    sorted_ids = jnp.zeros((R_PAD,), jnp.int32).at[p_arr].set(flat_tok[order])
    row_gate = jnp.zeros((R_PAD,), jnp.float32).at[p_arr].set(flat_g[order])
    ends = jnp.cumsum(padded).astype(jnp.int32)
    blk_base = jnp.arange(NUM_M_BLKS, dtype=jnp.int32) * M_BLK
    block_expert = jnp.clip(
        jnp.searchsorted(ends, blk_base, side="right").astype(jnp.int32),
        0, NUM_EXPERTS - 1)
    pos0 = pos_of_entry[0::TOPK]
    pos1 = pos_of_entry[1::TOPK]
    return sorted_ids, row_gate, block_expert, pos0, pos1


def kernel(x, ln1_g, ln1_b, ln2_g, ln2_b, Wq, bq, Wk, bk, Wv, bv,
           Wo, bo, Wr, W1, b1, W2, b2):
    B, S, D = x.shape
    x2d = x.reshape(S, D)
    v1 = lambda a: a.reshape(1, D)
    q, k, v = _qkv(x2d, v1(ln1_g), v1(ln1_b), Wq, v1(bq), Wk, v1(bk), Wv, v1(bv))
    H, dh = NUM_HEADS, D // NUM_HEADS
    to3 = lambda a: a.reshape(S, H, dh).transpose(1, 0, 2)
    o3 = _attention(to3(q), to3(k), to3(v))
    o = o3.transpose(1, 0, 2).reshape(S, D)
    x2, t, topi, gates, counts, psum = _router(
        x2d, o, Wo, v1(bo), v1(ln2_g), v1(ln2_b), Wr)
    sorted_ids, row_gate, block_expert, pos0, pos1 = _routing_metadata(
        topi, gates, counts)
    xs = _sc_gather_rows(t, sorted_ids, R_PAD, D)
    ys = _moe_grouped(block_expert, xs, W1, b1, W2, b2,
                      row_gate.reshape(R_PAD, 1))
    poscat = jnp.concatenate([pos0, pos1])
    rows = _sc_gather_rows(ys, poscat, N_ENTRY, D)
    out, lb = _final(x2, rows, counts, psum)
    return (out.reshape(B, S, D), lb.reshape(()))


# speedup vs baseline: 2.4170x; 1.0512x over previous
import functools

import jax
import jax.numpy as jnp
from jax import lax
from jax.experimental import pallas as pl
from jax.experimental.pallas import tpu as pltpu
from jax.experimental.pallas import tpu_sc as plsc

EMBED_DIM = 768
NUM_HEADS = 12
NUM_EXPERTS = 8
TOPK = 2
LB_W = 0.01
FF = EMBED_DIM * 4

S_BLK = 256
Q_BLK = 1024
M_BLK = 128
F_BLK = 768
N_TOK = 2048
N_ENTRY = N_TOK * TOPK
R_PAD = N_ENTRY + NUM_EXPERTS * M_BLK
NUM_M_BLKS = R_PAD // M_BLK
NUM_F_BLKS = FF // F_BLK

_SC_NC = 2
_SC_NS = 16
_SC_NW = _SC_NC * _SC_NS


def _sc_gather_rows(table, idx, nrows, ncols):
    b_per_w = nrows // _SC_NW
    mesh = plsc.VectorSubcoreMesh(core_axis_name="c", subcore_axis_name="s")

    @functools.partial(
        pl.kernel, mesh=mesh,
        out_type=jax.ShapeDtypeStruct((nrows, ncols), jnp.float32),
        compiler_params=pltpu.CompilerParams(use_tc_tiling_on_sc=True),
        scratch_types=[
            pltpu.VMEM((b_per_w,), jnp.int32),
            pltpu.VMEM((b_per_w, ncols), jnp.float32),
            pltpu.SemaphoreType.DMA,
        ],
    )
    def k(table_hbm, idx_hbm, out_hbm, idx_v, rows_v, sem):
        wid = lax.axis_index("s") * _SC_NC + lax.axis_index("c")
        base = wid * b_per_w
        pltpu.sync_copy(idx_hbm.at[pl.ds(base, b_per_w)], idx_v)
        pltpu.async_copy(table_hbm.at[idx_v], rows_v, sem).wait()
        pltpu.sync_copy(rows_v, out_hbm.at[pl.ds(base, b_per_w)])

    return k(table, idx)


def _ln(x, g, b):
    m = jnp.mean(x, -1, keepdims=True)
    v = jnp.mean((x - m) * (x - m), -1, keepdims=True)
    return (x - m) * lax.rsqrt(v + 1e-5) * g + b


def _qkv_body(x_ref, g_ref, b_ref, wq_ref, bq_ref, wk_ref, bk_ref,
              wv_ref, bv_ref, q_out, k_out, v_out):
    h = _ln(x_ref[...], g_ref[...], b_ref[...])
    q_out[...] = jnp.dot(h, wq_ref[...], preferred_element_type=jnp.float32) + bq_ref[...]
    k_out[...] = jnp.dot(h, wk_ref[...], preferred_element_type=jnp.float32) + bk_ref[...]
    v_out[...] = jnp.dot(h, wv_ref[...], preferred_element_type=jnp.float32) + bv_ref[...]


def _qkv(x, ln1_g, ln1_b, Wq, bq, Wk, bk, Wv, bv):
    D = EMBED_DIM
    nblk = N_TOK // S_BLK
    row = pl.BlockSpec((S_BLK, D), lambda i: (i, 0))
    full = pl.BlockSpec((D, D), lambda i: (0, 0))
    vec = pl.BlockSpec((1, D), lambda i: (0, 0))
    out = jax.ShapeDtypeStruct((N_TOK, D), jnp.float32)
    return pl.pallas_call(
        _qkv_body,
        grid=(nblk,),
        in_specs=[row, vec, vec, full, vec, full, vec, full, vec],
        out_specs=[row, row, row],
        out_shape=[out, out, out],
    )(x, ln1_g, ln1_b, Wq, bq, Wk, bk, Wv, bv)


def _attn_body(q_ref, k_ref, v_ref, o_ref):
    dh = EMBED_DIM // NUM_HEADS
    s = lax.dot_general(q_ref[0], k_ref[0],
                        (((1,), (1,)), ((), ())),
                        preferred_element_type=jnp.float32)
    s = s * (1.0 / jnp.sqrt(jnp.float32(dh)))
    m = jnp.max(s, axis=-1, keepdims=True)
    p = jnp.exp(s - m)
    o = jnp.dot(p, v_ref[0], preferred_element_type=jnp.float32)
    o_ref[0] = o * (1.0 / jnp.sum(p, axis=-1, keepdims=True))


def _attention(q3, k3, v3):
    dh = EMBED_DIM // NUM_HEADS
    nq = N_TOK // Q_BLK
    qspec = pl.BlockSpec((1, Q_BLK, dh), lambda h, i: (h, i, 0))
    kvspec = pl.BlockSpec((1, N_TOK, dh), lambda h, i: (h, 0, 0))
    return pl.pallas_call(
        _attn_body,
        grid=(NUM_HEADS, nq),
        in_specs=[qspec, kvspec, kvspec],
        out_specs=qspec,
        out_shape=jax.ShapeDtypeStruct((NUM_HEADS, N_TOK, dh), jnp.float32),
    )(q3, k3, v3)


def _cumsum_rows(x):
    n = x.shape[0]
    s = 1
    while s < n:
        x = x + jnp.concatenate([jnp.zeros((s, x.shape[1]), x.dtype), x[:-s]], 0)
        s *= 2
    return x


def _router_body(x_ref, o_ref, wo_ref, bo_ref, g_ref, b_ref, wr_ref,
                 x2_out, t_out, topi_out, gates_out,
                 counts0_out, counts1_out, psum_out, rank_out):
    i = pl.program_id(0)
    E = NUM_EXPERTS
    x2 = x_ref[...] + jnp.dot(o_ref[...], wo_ref[...],
                              preferred_element_type=jnp.float32) + bo_ref[...]
    x2_out[...] = x2
    t = _ln(x2, g_ref[...], b_ref[...])
    t_out[...] = t
    logits = jnp.dot(t, wr_ref[...], preferred_element_type=jnp.float32)
    lm = jnp.max(logits, axis=-1, keepdims=True)
    pe = jnp.exp(logits - lm)
    probs = pe / jnp.sum(pe, axis=-1, keepdims=True)
    iota = lax.broadcasted_iota(jnp.int32, probs.shape, 1)
    m1 = jnp.max(probs, axis=-1, keepdims=True)
    i1 = jnp.min(jnp.where(probs == m1, iota, E), axis=-1, keepdims=True)
    probs2 = jnp.where(iota == i1, -1.0, probs)
    m2 = jnp.max(probs2, axis=-1, keepdims=True)
    i2 = jnp.min(jnp.where(probs2 == m2, iota, E), axis=-1, keepdims=True)
    denom = m1 + m2
    topi_out[...] = jnp.concatenate([i1, i2], axis=1)
    gates_out[...] = jnp.concatenate([m1 / denom, m2 / denom], axis=1)
    onehot0 = (iota == i1).astype(jnp.float32)
    onehot1 = (iota == i2).astype(jnp.float32)
    ps = jnp.sum(probs, axis=0, keepdims=True)

    @pl.when(i == 0)
    def _():
        counts0_out[...] = jnp.zeros_like(counts0_out)
        counts1_out[...] = jnp.zeros_like(counts1_out)
        psum_out[...] = jnp.zeros_like(psum_out)

    off0 = counts0_out[...]
    off1 = counts1_out[...]
    exc0 = _cumsum_rows(onehot0) - onehot0
    exc1 = _cumsum_rows(onehot1) - onehot1
    rank0 = jnp.sum(onehot0 * (off0 + exc0), axis=1, keepdims=True)
    rank1 = jnp.sum(onehot1 * (off1 + exc1), axis=1, keepdims=True)
    rank_out[...] = jnp.concatenate([rank0, rank1], 1).astype(jnp.int32)
    counts0_out[...] = off0 + jnp.sum(onehot0, axis=0, keepdims=True)
    counts1_out[...] = off1 + jnp.sum(onehot1, axis=0, keepdims=True)
    psum_out[...] += ps


def _router(x, o, Wo, bo, ln2_g, ln2_b, Wr):
    D, E = EMBED_DIM, NUM_EXPERTS
    nblk = N_TOK // S_BLK
    row = pl.BlockSpec((S_BLK, D), lambda i: (i, 0))
    full = pl.BlockSpec((D, D), lambda i: (0, 0))
    vec = pl.BlockSpec((1, D), lambda i: (0, 0))
    wr = pl.BlockSpec((D, E), lambda i: (0, 0))
    two = pl.BlockSpec((S_BLK, TOPK), lambda i: (i, 0))
    acc = pl.BlockSpec((1, E), lambda i: (0, 0))
    return pl.pallas_call(
        _router_body,
        grid=(nblk,),
        in_specs=[row, row, full, vec, vec, vec, wr],
        out_specs=[row, row, two, two, acc, acc, acc, two],
        out_shape=[
            jax.ShapeDtypeStruct((N_TOK, D), jnp.float32),
            jax.ShapeDtypeStruct((N_TOK, D), jnp.float32),
            jax.ShapeDtypeStruct((N_TOK, TOPK), jnp.int32),
            jax.ShapeDtypeStruct((N_TOK, TOPK), jnp.float32),
            jax.ShapeDtypeStruct((1, E), jnp.float32),
            jax.ShapeDtypeStruct((1, E), jnp.float32),
            jax.ShapeDtypeStruct((1, E), jnp.float32),
            jax.ShapeDtypeStruct((N_TOK, TOPK), jnp.int32),
        ],
    )(x, o, Wo, bo, ln2_g, ln2_b, Wr)


def _positions_body(topi_ref, rank_ref, c0_ref, c1_ref, pos_out, be_out):
    i = pl.program_id(0)
    E = NUM_EXPERTS
    counts = c0_ref[...] + c1_ref[...]
    padded = jnp.ceil(counts * (1.0 / M_BLK)) * M_BLK
    inc = padded
    s = 1
    while s < E:
        inc = inc + jnp.concatenate(
            [jnp.zeros((1, s), jnp.float32), inc[:, :-s]], 1)
        s *= 2
    start = inc - padded
    iota = lax.broadcasted_iota(jnp.int32, (S_BLK, E), 1)
    oh0 = (iota == topi_ref[:, 0:1]).astype(jnp.float32)
    oh1 = (iota == topi_ref[:, 1:2]).astype(jnp.float32)
    base0 = jnp.sum(oh0 * start, axis=1, keepdims=True)
    base1 = jnp.sum(oh1 * (start + c0_ref[...]), axis=1, keepdims=True)
    rank = rank_ref[...].astype(jnp.float32)
    pos0 = base0 + rank[:, 0:1]
    pos1 = base1 + rank[:, 1:2]
    pos_out[...] = jnp.concatenate([pos0, pos1], 1).astype(jnp.int32)

    @pl.when(i == 0)
    def _():
        ends = start + padded
        blk = lax.broadcasted_iota(
            jnp.int32, (1, NUM_M_BLKS), 1).astype(jnp.float32) * jnp.float32(M_BLK)
        acc = jnp.zeros((1, NUM_M_BLKS), jnp.float32)
        for e in range(E):
            acc += (blk >= ends[:, e:e + 1]).astype(jnp.float32)
        be_out[...] = jnp.minimum(acc, E - 1).astype(jnp.int32)


def _positions(topi, rank, counts0, counts1):
    E = NUM_EXPERTS
    nblk = N_TOK // S_BLK
    two = pl.BlockSpec((S_BLK, TOPK), lambda i: (i, 0))
    acc = pl.BlockSpec((1, E), lambda i: (0, 0))
    bes = pl.BlockSpec((1, NUM_M_BLKS), lambda i: (0, 0))
    return pl.pallas_call(
        _positions_body,
        grid=(nblk,),
        in_specs=[two, two, acc, acc],
        out_specs=[two, bes],
        out_shape=[
            jax.ShapeDtypeStruct((N_TOK, TOPK), jnp.int32),
            jax.ShapeDtypeStruct((1, NUM_M_BLKS), jnp.int32),
        ],
    )(topi, rank, counts0, counts1)


def _moe_body(be_ref, xs_ref, w1_ref, b1_ref, w2_ref, b2_ref, gate_ref, out_ref):
    h = jnp.dot(xs_ref[...], w1_ref[0], preferred_element_type=jnp.float32)
    h = h + b1_ref[0]
    h = 0.5 * h * (1.0 + lax.erf(h * jnp.float32(0.7071067811865476)))
    y = jnp.dot(h, w2_ref[0], preferred_element_type=jnp.float32)
    out_ref[...] = (y + b2_ref[0]) * gate_ref[...]


def _moe_grouped(block_expert, xs, W1, b1, W2, b2, row_gate):
    D, F = EMBED_DIM, FF
    grid_spec = pltpu.PrefetchScalarGridSpec(
        num_scalar_prefetch=1,
        grid=(NUM_M_BLKS,),
        in_specs=[
            pl.BlockSpec((M_BLK, D), lambda i, be: (i, 0)),
            pl.BlockSpec((1, D, F), lambda i, be: (be[i], 0, 0)),
            pl.BlockSpec((1, 1, F), lambda i, be: (be[i], 0, 0)),
            pl.BlockSpec((1, F, D), lambda i, be: (be[i], 0, 0)),
            pl.BlockSpec((1, 1, D), lambda i, be: (be[i], 0, 0)),
            pl.BlockSpec((M_BLK, 1), lambda i, be: (i, 0)),
        ],
        out_specs=pl.BlockSpec((M_BLK, D), lambda i, be: (i, 0)),
    )
    return pl.pallas_call(
        _moe_body,
        grid_spec=grid_spec,
        out_shape=jax.ShapeDtypeStruct((R_PAD, D), jnp.float32),
    )(block_expert, xs, W1, b1.reshape(NUM_EXPERTS, 1, F),
      W2, b2.reshape(NUM_EXPERTS, 1, D), row_gate)


def _final_body(x2_ref, r0_ref, r1_ref, c0_ref, c1_ref, psum_ref,
                out_ref, lb_ref):
    i = pl.program_id(0)
    out_ref[...] = x2_ref[...] + r0_ref[...] + r1_ref[...]

    @pl.when(i == 0)
    def _():
        counts = c0_ref[...] + c1_ref[...]
        frac = counts / jnp.float32(N_TOK * TOPK)
        pmean = psum_ref[...] / jnp.float32(N_TOK)
        lb_ref[...] = (LB_W * NUM_EXPERTS) * jnp.sum(
            frac * pmean, keepdims=True).reshape(1, 1)


def _final(x2, rows, counts0, counts1, psum):
    D, E = EMBED_DIM, NUM_EXPERTS
    nblk = N_TOK // S_BLK
    row = pl.BlockSpec((S_BLK, D), lambda i: (i, 0))
    row1 = pl.BlockSpec((S_BLK, D), lambda i: (i + nblk, 0))
    acc = pl.BlockSpec((1, E), lambda i: (0, 0))
    one = pl.BlockSpec((1, 1), lambda i: (0, 0))
    return pl.pallas_call(
        _final_body,
        grid=(nblk,),
        in_specs=[row, row, row1, acc, acc, acc],
        out_specs=[row, one],
        out_shape=[
            jax.ShapeDtypeStruct((N_TOK, D), jnp.float32),
            jax.ShapeDtypeStruct((1, 1), jnp.float32),
        ],
    )(x2, rows, rows, counts0, counts1, psum)


def _dispatch_tables(pos, gates):
    poscat = jnp.concatenate([pos[:, 0], pos[:, 1]])
    gcat = jnp.concatenate([gates[:, 0], gates[:, 1]])
    tok = jnp.arange(N_TOK, dtype=jnp.int32)
    tokcat = jnp.concatenate([tok, tok])
    sorted_ids = jnp.zeros((R_PAD,), jnp.int32).at[poscat].set(tokcat)
    row_gate = jnp.zeros((R_PAD,), jnp.float32).at[poscat].set(gcat)
    return sorted_ids, row_gate, poscat


def kernel(x, ln1_g, ln1_b, ln2_g, ln2_b, Wq, bq, Wk, bk, Wv, bv,
           Wo, bo, Wr, W1, b1, W2, b2):
    B, S, D = x.shape
    x2d = x.reshape(S, D)
    v1 = lambda a: a.reshape(1, D)
    q, k, v = _qkv(x2d, v1(ln1_g), v1(ln1_b), Wq, v1(bq), Wk, v1(bk), Wv, v1(bv))
    H, dh = NUM_HEADS, D // NUM_HEADS
    to3 = lambda a: a.reshape(S, H, dh).transpose(1, 0, 2)
    o3 = _attention(to3(q), to3(k), to3(v))
    o = o3.transpose(1, 0, 2).reshape(S, D)
    x2, t, topi, gates, counts0, counts1, psum, rank = _router(
        x2d, o, Wo, v1(bo), v1(ln2_g), v1(ln2_b), Wr)
    pos, be = _positions(topi, rank, counts0, counts1)
    sorted_ids, row_gate, poscat = _dispatch_tables(pos, gates)
    xs = _sc_gather_rows(t, sorted_ids, R_PAD, D)
    ys = _moe_grouped(be.reshape(NUM_M_BLKS), xs, W1, b1, W2, b2,
                      row_gate.reshape(R_PAD, 1))
    rows = _sc_gather_rows(ys, poscat, N_ENTRY, D)
    out, lb = _final(x2, rows, counts0, counts1, psum)
    return (out.reshape(B, S, D), lb.reshape(()))


# speedup vs baseline: 2.7158x; 1.1236x over previous
import functools

import jax
import jax.numpy as jnp
from jax import lax
from jax.experimental import pallas as pl
from jax.experimental.pallas import tpu as pltpu
from jax.experimental.pallas import tpu_sc as plsc

EMBED_DIM = 768
NUM_HEADS = 12
NUM_EXPERTS = 8
TOPK = 2
LB_W = 0.01
FF = EMBED_DIM * 4

S_BLK = 256
Q_BLK = 1024
M_BLK = 128
F_BLK = 768
N_TOK = 2048
N_ENTRY = N_TOK * TOPK
R_PAD = N_ENTRY + NUM_EXPERTS * M_BLK
NUM_M_BLKS = R_PAD // M_BLK
NUM_F_BLKS = FF // F_BLK

_SC_NC = 2
_SC_NS = 16
_SC_NW = _SC_NC * _SC_NS


def _sc_gather_rows(table, idx, nrows, ncols):
    b_per_w = nrows // _SC_NW
    mesh = plsc.VectorSubcoreMesh(core_axis_name="c", subcore_axis_name="s")

    @functools.partial(
        pl.kernel, mesh=mesh,
        out_type=jax.ShapeDtypeStruct((nrows, ncols), jnp.float32),
        compiler_params=pltpu.CompilerParams(use_tc_tiling_on_sc=True),
        scratch_types=[
            pltpu.VMEM((b_per_w,), jnp.int32),
            pltpu.VMEM((b_per_w, ncols), jnp.float32),
            pltpu.SemaphoreType.DMA,
        ],
    )
    def k(table_hbm, idx_hbm, out_hbm, idx_v, rows_v, sem):
        wid = lax.axis_index("s") * _SC_NC + lax.axis_index("c")
        base = wid * b_per_w
        pltpu.sync_copy(idx_hbm.at[pl.ds(base, b_per_w)], idx_v)
        pltpu.async_copy(table_hbm.at[idx_v], rows_v, sem).wait()
        pltpu.sync_copy(rows_v, out_hbm.at[pl.ds(base, b_per_w)])

    return k(table, idx)


def _ln(x, g, b):
    m = jnp.mean(x, -1, keepdims=True)
    v = jnp.mean((x - m) * (x - m), -1, keepdims=True)
    return (x - m) * lax.rsqrt(v + 1e-5) * g + b


def _qkv_body(x_ref, g_ref, b_ref, wq_ref, bq_ref, wk_ref, bk_ref,
              wv_ref, bv_ref, q_out, k_out, v_out):
    h = _ln(x_ref[...], g_ref[...], b_ref[...])
    q_out[...] = jnp.dot(h, wq_ref[...], preferred_element_type=jnp.float32) + bq_ref[...]
    k_out[...] = jnp.dot(h, wk_ref[...], preferred_element_type=jnp.float32) + bk_ref[...]
    v_out[...] = jnp.dot(h, wv_ref[...], preferred_element_type=jnp.float32) + bv_ref[...]


def _qkv(x, ln1_g, ln1_b, Wq, bq, Wk, bk, Wv, bv):
    D = EMBED_DIM
    nblk = N_TOK // S_BLK
    row = pl.BlockSpec((S_BLK, D), lambda i: (i, 0))
    full = pl.BlockSpec((D, D), lambda i: (0, 0))
    vec = pl.BlockSpec((1, D), lambda i: (0, 0))
    out = jax.ShapeDtypeStruct((N_TOK, D), jnp.float32)
    return pl.pallas_call(
        _qkv_body,
        grid=(nblk,),
        in_specs=[row, vec, vec, full, vec, full, vec, full, vec],
        out_specs=[row, row, row],
        out_shape=[out, out, out],
    )(x, ln1_g, ln1_b, Wq, bq, Wk, bk, Wv, bv)


def _attn_body(q_ref, k_ref, v_ref, o_ref):
    dh = EMBED_DIM // NUM_HEADS
    s = lax.dot_general(q_ref[0], k_ref[0],
                        (((1,), (1,)), ((), ())),
                        preferred_element_type=jnp.float32)
    s = s * (1.0 / jnp.sqrt(jnp.float32(dh)))
    m = jnp.max(s, axis=-1, keepdims=True)
    p = jnp.exp(s - m)
    o = jnp.dot(p, v_ref[0], preferred_element_type=jnp.float32)
    o_ref[0] = o * (1.0 / jnp.sum(p, axis=-1, keepdims=True))


def _attention(q3, k3, v3):
    dh = EMBED_DIM // NUM_HEADS
    nq = N_TOK // Q_BLK
    qspec = pl.BlockSpec((1, Q_BLK, dh), lambda h, i: (h, i, 0))
    kvspec = pl.BlockSpec((1, N_TOK, dh), lambda h, i: (h, 0, 0))
    return pl.pallas_call(
        _attn_body,
        grid=(NUM_HEADS, nq),
        in_specs=[qspec, kvspec, kvspec],
        out_specs=qspec,
        out_shape=jax.ShapeDtypeStruct((NUM_HEADS, N_TOK, dh), jnp.float32),
    )(q3, k3, v3)


def _cumsum_rows(x):
    n = x.shape[0]
    s = 1
    while s < n:
        x = x + jnp.concatenate([jnp.zeros((s, x.shape[1]), x.dtype), x[:-s]], 0)
        s *= 2
    return x


def _router_body(x_ref, o_ref, wo_ref, bo_ref, g_ref, b_ref, wr_ref,
                 x2_out, t_out, topi_out, gates_out,
                 counts0_out, counts1_out, psum_out, rank_out):
    i = pl.program_id(0)
    E = NUM_EXPERTS
    x2 = x_ref[...] + jnp.dot(o_ref[...], wo_ref[...],
                              preferred_element_type=jnp.float32) + bo_ref[...]
    x2_out[...] = x2
    t = _ln(x2, g_ref[...], b_ref[...])
    t_out[...] = t
    logits = jnp.dot(t, wr_ref[...], preferred_element_type=jnp.float32)
    lm = jnp.max(logits, axis=-1, keepdims=True)
    pe = jnp.exp(logits - lm)
    probs = pe / jnp.sum(pe, axis=-1, keepdims=True)
    iota = lax.broadcasted_iota(jnp.int32, probs.shape, 1)
    m1 = jnp.max(probs, axis=-1, keepdims=True)
    i1 = jnp.min(jnp.where(probs == m1, iota, E), axis=-1, keepdims=True)
    probs2 = jnp.where(iota == i1, -1.0, probs)
    m2 = jnp.max(probs2, axis=-1, keepdims=True)
    i2 = jnp.min(jnp.where(probs2 == m2, iota, E), axis=-1, keepdims=True)
    denom = m1 + m2
    topi_out[...] = jnp.concatenate([i1, i2], axis=1)
    gates_out[...] = jnp.concatenate([m1 / denom, m2 / denom], axis=1)
    onehot0 = (iota == i1).astype(jnp.float32)
    onehot1 = (iota == i2).astype(jnp.float32)
    ps = jnp.sum(probs, axis=0, keepdims=True)

    @pl.when(i == 0)
    def _():
        counts0_out[...] = jnp.zeros_like(counts0_out)
        counts1_out[...] = jnp.zeros_like(counts1_out)
        psum_out[...] = jnp.zeros_like(psum_out)

    off0 = counts0_out[...]
    off1 = counts1_out[...]
    exc0 = _cumsum_rows(onehot0) - onehot0
    exc1 = _cumsum_rows(onehot1) - onehot1
    rank0 = jnp.sum(onehot0 * (off0 + exc0), axis=1, keepdims=True)
    rank1 = jnp.sum(onehot1 * (off1 + exc1), axis=1, keepdims=True)
    rank_out[...] = jnp.concatenate([rank0, rank1], 1).astype(jnp.int32)
    counts0_out[...] = off0 + jnp.sum(onehot0, axis=0, keepdims=True)
    counts1_out[...] = off1 + jnp.sum(onehot1, axis=0, keepdims=True)
    psum_out[...] += ps


def _router(x, o, Wo, bo, ln2_g, ln2_b, Wr):
    D, E = EMBED_DIM, NUM_EXPERTS
    nblk = N_TOK // S_BLK
    row = pl.BlockSpec((S_BLK, D), lambda i: (i, 0))
    full = pl.BlockSpec((D, D), lambda i: (0, 0))
    vec = pl.BlockSpec((1, D), lambda i: (0, 0))
    wr = pl.BlockSpec((D, E), lambda i: (0, 0))
    two = pl.BlockSpec((S_BLK, TOPK), lambda i: (i, 0))
    acc = pl.BlockSpec((1, E), lambda i: (0, 0))
    return pl.pallas_call(
        _router_body,
        grid=(nblk,),
        in_specs=[row, row, full, vec, vec, vec, wr],
        out_specs=[row, row, two, two, acc, acc, acc, two],
        out_shape=[
            jax.ShapeDtypeStruct((N_TOK, D), jnp.float32),
            jax.ShapeDtypeStruct((N_TOK, D), jnp.float32),
            jax.ShapeDtypeStruct((N_TOK, TOPK), jnp.int32),
            jax.ShapeDtypeStruct((N_TOK, TOPK), jnp.float32),
            jax.ShapeDtypeStruct((1, E), jnp.float32),
            jax.ShapeDtypeStruct((1, E), jnp.float32),
            jax.ShapeDtypeStruct((1, E), jnp.float32),
            jax.ShapeDtypeStruct((N_TOK, TOPK), jnp.int32),
        ],
    )(x, o, Wo, bo, ln2_g, ln2_b, Wr)


def _positions_body(topi_ref, rank_ref, c0_ref, c1_ref, pos_out, be_out):
    i = pl.program_id(0)
    E = NUM_EXPERTS
    counts = c0_ref[...] + c1_ref[...]
    padded = jnp.ceil(counts * (1.0 / M_BLK)) * M_BLK
    inc = padded
    s = 1
    while s < E:
        inc = inc + jnp.concatenate(
            [jnp.zeros((1, s), jnp.float32), inc[:, :-s]], 1)
        s *= 2
    start = inc - padded
    iota = lax.broadcasted_iota(jnp.int32, (S_BLK, E), 1)
    oh0 = (iota == topi_ref[:, 0:1]).astype(jnp.float32)
    oh1 = (iota == topi_ref[:, 1:2]).astype(jnp.float32)
    base0 = jnp.sum(oh0 * start, axis=1, keepdims=True)
    base1 = jnp.sum(oh1 * (start + c0_ref[...]), axis=1, keepdims=True)
    rank = rank_ref[...].astype(jnp.float32)
    pos0 = base0 + rank[:, 0:1]
    pos1 = base1 + rank[:, 1:2]
    pos_out[...] = jnp.concatenate([pos0, pos1], 1).astype(jnp.int32)

    @pl.when(i == 0)
    def _():
        ends = start + padded
        blk = lax.broadcasted_iota(
            jnp.int32, (1, NUM_M_BLKS), 1).astype(jnp.float32) * jnp.float32(M_BLK)
        acc = jnp.zeros((1, NUM_M_BLKS), jnp.float32)
        for e in range(E):
            acc += (blk >= ends[:, e:e + 1]).astype(jnp.float32)
        be_out[...] = jnp.minimum(acc, E - 1).astype(jnp.int32)


def _positions(topi, rank, counts0, counts1):
    E = NUM_EXPERTS
    nblk = N_TOK // S_BLK
    two = pl.BlockSpec((S_BLK, TOPK), lambda i: (i, 0))
    acc = pl.BlockSpec((1, E), lambda i: (0, 0))
    bes = pl.BlockSpec((1, NUM_M_BLKS), lambda i: (0, 0))
    return pl.pallas_call(
        _positions_body,
        grid=(nblk,),
        in_specs=[two, two, acc, acc],
        out_specs=[two, bes],
        out_shape=[
            jax.ShapeDtypeStruct((N_TOK, TOPK), jnp.int32),
            jax.ShapeDtypeStruct((1, NUM_M_BLKS), jnp.int32),
        ],
    )(topi, rank, counts0, counts1)


def _moe_body(be_ref, xs_ref, w1_ref, b1_ref, w2_ref, b2_ref, out_ref):
    h = jnp.dot(xs_ref[...], w1_ref[0], preferred_element_type=jnp.float32)
    h = h + b1_ref[0]
    h = 0.5 * h * (1.0 + lax.erf(h * jnp.float32(0.7071067811865476)))
    y = jnp.dot(h, w2_ref[0], preferred_element_type=jnp.float32)
    out_ref[...] = y + b2_ref[0]


def _moe_grouped(block_expert, xs, W1, b1, W2, b2):
    D, F = EMBED_DIM, FF
    grid_spec = pltpu.PrefetchScalarGridSpec(
        num_scalar_prefetch=1,
        grid=(NUM_M_BLKS,),
        in_specs=[
            pl.BlockSpec((M_BLK, D), lambda i, be: (i, 0)),
            pl.BlockSpec((1, D, F), lambda i, be: (be[i], 0, 0)),
            pl.BlockSpec((1, 1, F), lambda i, be: (be[i], 0, 0)),
            pl.BlockSpec((1, F, D), lambda i, be: (be[i], 0, 0)),
            pl.BlockSpec((1, 1, D), lambda i, be: (be[i], 0, 0)),
        ],
        out_specs=pl.BlockSpec((M_BLK, D), lambda i, be: (i, 0)),
    )
    return pl.pallas_call(
        _moe_body,
        grid_spec=grid_spec,
        out_shape=jax.ShapeDtypeStruct((R_PAD, D), jnp.float32),
    )(block_expert, xs, W1, b1.reshape(NUM_EXPERTS, 1, F),
      W2, b2.reshape(NUM_EXPERTS, 1, D))


def _final_body(x2_ref, r0_ref, r1_ref, gates_ref, c0_ref, c1_ref, psum_ref,
                out_ref, lb_ref):
    i = pl.program_id(0)
    out_ref[...] = (x2_ref[...] + gates_ref[:, 0:1] * r0_ref[...]
                    + gates_ref[:, 1:2] * r1_ref[...])

    @pl.when(i == 0)
    def _():
        counts = c0_ref[...] + c1_ref[...]
        frac = counts / jnp.float32(N_TOK * TOPK)
        pmean = psum_ref[...] / jnp.float32(N_TOK)
        lb_ref[...] = (LB_W * NUM_EXPERTS) * jnp.sum(
            frac * pmean, keepdims=True).reshape(1, 1)


def _final(x2, rows, gates, counts0, counts1, psum):
    D, E = EMBED_DIM, NUM_EXPERTS
    nblk = N_TOK // S_BLK
    row = pl.BlockSpec((S_BLK, D), lambda i: (i, 0))
    row1 = pl.BlockSpec((S_BLK, D), lambda i: (i + nblk, 0))
    two = pl.BlockSpec((S_BLK, TOPK), lambda i: (i, 0))
    acc = pl.BlockSpec((1, E), lambda i: (0, 0))
    one = pl.BlockSpec((1, 1), lambda i: (0, 0))
    return pl.pallas_call(
        _final_body,
        grid=(nblk,),
        in_specs=[row, row, row1, two, acc, acc, acc],
        out_specs=[row, one],
        out_shape=[
            jax.ShapeDtypeStruct((N_TOK, D), jnp.float32),
            jax.ShapeDtypeStruct((1, 1), jnp.float32),
        ],
    )(x2, rows, rows, gates, counts0, counts1, psum)


def _dispatch_tables(pos):
    poscat = jnp.concatenate([pos[:, 0], pos[:, 1]])
    tok = jnp.arange(N_TOK, dtype=jnp.int32)
    tokcat = jnp.concatenate([tok, tok])
    base = jnp.arange(R_PAD, dtype=jnp.int32) % N_TOK
    sorted_ids = base.at[poscat].set(tokcat)
    return sorted_ids, poscat


def kernel(x, ln1_g, ln1_b, ln2_g, ln2_b, Wq, bq, Wk, bk, Wv, bv,
           Wo, bo, Wr, W1, b1, W2, b2):
    B, S, D = x.shape
    x2d = x.reshape(S, D)
    v1 = lambda a: a.reshape(1, D)
    q, k, v = _qkv(x2d, v1(ln1_g), v1(ln1_b), Wq, v1(bq), Wk, v1(bk), Wv, v1(bv))
    H, dh = NUM_HEADS, D // NUM_HEADS
    to3 = lambda a: a.reshape(S, H, dh).transpose(1, 0, 2)
    o3 = _attention(to3(q), to3(k), to3(v))
    o = o3.transpose(1, 0, 2).reshape(S, D)
    x2, t, topi, gates, counts0, counts1, psum, rank = _router(
        x2d, o, Wo, v1(bo), v1(ln2_g), v1(ln2_b), Wr)
    pos, be = _positions(topi, rank, counts0, counts1)
    sorted_ids, poscat = _dispatch_tables(pos)
    xs = _sc_gather_rows(t, sorted_ids, R_PAD, D)
    ys = _moe_grouped(be.reshape(NUM_M_BLKS), xs, W1, b1, W2, b2)
    rows = _sc_gather_rows(ys, poscat, N_ENTRY, D)
    out, lb = _final(x2, rows, gates, counts0, counts1, psum)
    return (out.reshape(B, S, D), lb.reshape(()))


# speedup vs baseline: 2.8298x; 1.0420x over previous
import functools

import jax
import jax.numpy as jnp
from jax import lax
from jax.experimental import pallas as pl
from jax.experimental.pallas import tpu as pltpu
from jax.experimental.pallas import tpu_sc as plsc

EMBED_DIM = 768
NUM_HEADS = 12
NUM_EXPERTS = 8
TOPK = 2
LB_W = 0.01
FF = EMBED_DIM * 4

S_BLK = 256
Q_BLK = 1024
M_BLK = 128
F_BLK = 768
N_TOK = 2048
N_ENTRY = N_TOK * TOPK
R_PAD = N_ENTRY + NUM_EXPERTS * M_BLK
NUM_M_BLKS = R_PAD // M_BLK
NUM_F_BLKS = FF // F_BLK

_SC_NC = 2
_SC_NS = 16
_SC_NW = _SC_NC * _SC_NS


def _sc_gather_rows(table, idx, nrows, ncols):
    b_per_w = nrows // _SC_NW
    mesh = plsc.VectorSubcoreMesh(core_axis_name="c", subcore_axis_name="s")

    @functools.partial(
        pl.kernel, mesh=mesh,
        out_type=jax.ShapeDtypeStruct((nrows, ncols), jnp.float32),
        compiler_params=pltpu.CompilerParams(use_tc_tiling_on_sc=True),
        scratch_types=[
            pltpu.VMEM((b_per_w,), jnp.int32),
            pltpu.VMEM((b_per_w, ncols), jnp.float32),
            pltpu.SemaphoreType.DMA,
        ],
    )
    def k(table_hbm, idx_hbm, out_hbm, idx_v, rows_v, sem):
        wid = lax.axis_index("s") * _SC_NC + lax.axis_index("c")
        base = wid * b_per_w
        pltpu.sync_copy(idx_hbm.at[pl.ds(base, b_per_w)], idx_v)
        pltpu.async_copy(table_hbm.at[idx_v], rows_v, sem).wait()
        pltpu.sync_copy(rows_v, out_hbm.at[pl.ds(base, b_per_w)])

    return k(table, idx)


def _sc_dispatch_scatter(t, poscat):
    b_per_w = N_ENTRY // _SC_NW
    mesh = plsc.VectorSubcoreMesh(core_axis_name="c", subcore_axis_name="s")

    @functools.partial(
        pl.kernel, mesh=mesh,
        out_type=jax.ShapeDtypeStruct((R_PAD, EMBED_DIM), jnp.float32),
        compiler_params=pltpu.CompilerParams(use_tc_tiling_on_sc=True),
        scratch_types=[
            pltpu.VMEM((b_per_w,), jnp.int32),
            pltpu.VMEM((b_per_w, EMBED_DIM), jnp.float32),
            pltpu.SemaphoreType.DMA,
        ],
    )
    def k(t_hbm, pos_hbm, xs_hbm, pos_v, rows_v, sem):
        wid = lax.axis_index("s") * _SC_NC + lax.axis_index("c")
        base = wid * b_per_w
        tok0 = base - (base // N_TOK) * N_TOK
        pltpu.sync_copy(pos_hbm.at[pl.ds(base, b_per_w)], pos_v)
        pltpu.sync_copy(t_hbm.at[pl.ds(tok0, b_per_w)], rows_v)
        pltpu.async_copy(rows_v, xs_hbm.at[pos_v], sem).wait()

    return k(t, poscat)


def _ln(x, g, b):
    m = jnp.mean(x, -1, keepdims=True)
    v = jnp.mean((x - m) * (x - m), -1, keepdims=True)
    return (x - m) * lax.rsqrt(v + 1e-5) * g + b


def _qkv_body(x_ref, g_ref, b_ref, wq_ref, bq_ref, wk_ref, bk_ref,
              wv_ref, bv_ref, q_out, k_out, v_out):
    h = _ln(x_ref[...], g_ref[...], b_ref[...])
    q_out[...] = jnp.dot(h, wq_ref[...], preferred_element_type=jnp.float32) + bq_ref[...]
    k_out[...] = jnp.dot(h, wk_ref[...], preferred_element_type=jnp.float32) + bk_ref[...]
    v_out[...] = jnp.dot(h, wv_ref[...], preferred_element_type=jnp.float32) + bv_ref[...]


def _qkv(x, ln1_g, ln1_b, Wq, bq, Wk, bk, Wv, bv):
    D = EMBED_DIM
    nblk = N_TOK // S_BLK
    row = pl.BlockSpec((S_BLK, D), lambda i: (i, 0))
    full = pl.BlockSpec((D, D), lambda i: (0, 0))
    vec = pl.BlockSpec((1, D), lambda i: (0, 0))
    out = jax.ShapeDtypeStruct((N_TOK, D), jnp.float32)
    return pl.pallas_call(
        _qkv_body,
        grid=(nblk,),
        in_specs=[row, vec, vec, full, vec, full, vec, full, vec],
        out_specs=[row, row, row],
        out_shape=[out, out, out],
    )(x, ln1_g, ln1_b, Wq, bq, Wk, bk, Wv, bv)


def _attn_body(q_ref, k_ref, v_ref, o_ref):
    dh = EMBED_DIM // NUM_HEADS
    s = lax.dot_general(q_ref[0], k_ref[0],
                        (((1,), (1,)), ((), ())),
                        preferred_element_type=jnp.float32)
    s = s * (1.0 / jnp.sqrt(jnp.float32(dh)))
    m = jnp.max(s, axis=-1, keepdims=True)
    p = jnp.exp(s - m)
    o = jnp.dot(p, v_ref[0], preferred_element_type=jnp.float32)
    o_ref[0] = o * (1.0 / jnp.sum(p, axis=-1, keepdims=True))


def _attention(q3, k3, v3):
    dh = EMBED_DIM // NUM_HEADS
    nq = N_TOK // Q_BLK
    qspec = pl.BlockSpec((1, Q_BLK, dh), lambda h, i: (h, i, 0))
    kvspec = pl.BlockSpec((1, N_TOK, dh), lambda h, i: (h, 0, 0))
    return pl.pallas_call(
        _attn_body,
        grid=(NUM_HEADS, nq),
        in_specs=[qspec, kvspec, kvspec],
        out_specs=qspec,
        out_shape=jax.ShapeDtypeStruct((NUM_HEADS, N_TOK, dh), jnp.float32),
    )(q3, k3, v3)


def _cumsum_rows(x):
    n = x.shape[0]
    s = 1
    while s < n:
        x = x + jnp.concatenate([jnp.zeros((s, x.shape[1]), x.dtype), x[:-s]], 0)
        s *= 2
    return x


def _router_body(x_ref, o_ref, wo_ref, bo_ref, g_ref, b_ref, wr_ref,
                 x2_out, t_out, topi_out, gates_out,
                 counts0_out, counts1_out, psum_out, rank_out):
    i = pl.program_id(0)
    E = NUM_EXPERTS
    x2 = x_ref[...] + jnp.dot(o_ref[...], wo_ref[...],
                              preferred_element_type=jnp.float32) + bo_ref[...]
    x2_out[...] = x2
    t = _ln(x2, g_ref[...], b_ref[...])
    t_out[...] = t
    logits = jnp.dot(t, wr_ref[...], preferred_element_type=jnp.float32)
    lm = jnp.max(logits, axis=-1, keepdims=True)
    pe = jnp.exp(logits - lm)
    probs = pe / jnp.sum(pe, axis=-1, keepdims=True)
    iota = lax.broadcasted_iota(jnp.int32, probs.shape, 1)
    m1 = jnp.max(probs, axis=-1, keepdims=True)
    i1 = jnp.min(jnp.where(probs == m1, iota, E), axis=-1, keepdims=True)
    probs2 = jnp.where(iota == i1, -1.0, probs)
    m2 = jnp.max(probs2, axis=-1, keepdims=True)
    i2 = jnp.min(jnp.where(probs2 == m2, iota, E), axis=-1, keepdims=True)
    denom = m1 + m2
    topi_out[...] = jnp.concatenate([i1, i2], axis=1)
    gates_out[...] = jnp.concatenate([m1 / denom, m2 / denom], axis=1)
    onehot0 = (iota == i1).astype(jnp.float32)
    onehot1 = (iota == i2).astype(jnp.float32)
    ps = jnp.sum(probs, axis=0, keepdims=True)

    @pl.when(i == 0)
    def _():
        counts0_out[...] = jnp.zeros_like(counts0_out)
        counts1_out[...] = jnp.zeros_like(counts1_out)
        psum_out[...] = jnp.zeros_like(psum_out)

    off0 = counts0_out[...]
    off1 = counts1_out[...]
    exc0 = _cumsum_rows(onehot0) - onehot0
    exc1 = _cumsum_rows(onehot1) - onehot1
    rank0 = jnp.sum(onehot0 * (off0 + exc0), axis=1, keepdims=True)
    rank1 = jnp.sum(onehot1 * (off1 + exc1), axis=1, keepdims=True)
    rank_out[...] = jnp.concatenate([rank0, rank1], 1).astype(jnp.int32)
    counts0_out[...] = off0 + jnp.sum(onehot0, axis=0, keepdims=True)
    counts1_out[...] = off1 + jnp.sum(onehot1, axis=0, keepdims=True)
    psum_out[...] += ps


def _router(x, o, Wo, bo, ln2_g, ln2_b, Wr):
    D, E = EMBED_DIM, NUM_EXPERTS
    nblk = N_TOK // S_BLK
    row = pl.BlockSpec((S_BLK, D), lambda i: (i, 0))
    full = pl.BlockSpec((D, D), lambda i: (0, 0))
    vec = pl.BlockSpec((1, D), lambda i: (0, 0))
    wr = pl.BlockSpec((D, E), lambda i: (0, 0))
    two = pl.BlockSpec((S_BLK, TOPK), lambda i: (i, 0))
    acc = pl.BlockSpec((1, E), lambda i: (0, 0))
    return pl.pallas_call(
        _router_body,
        grid=(nblk,),
        in_specs=[row, row, full, vec, vec, vec, wr],
        out_specs=[row, row, two, two, acc, acc, acc, two],
        out_shape=[
            jax.ShapeDtypeStruct((N_TOK, D), jnp.float32),
            jax.ShapeDtypeStruct((N_TOK, D), jnp.float32),
            jax.ShapeDtypeStruct((N_TOK, TOPK), jnp.int32),
            jax.ShapeDtypeStruct((N_TOK, TOPK), jnp.float32),
            jax.ShapeDtypeStruct((1, E), jnp.float32),
            jax.ShapeDtypeStruct((1, E), jnp.float32),
            jax.ShapeDtypeStruct((1, E), jnp.float32),
            jax.ShapeDtypeStruct((N_TOK, TOPK), jnp.int32),
        ],
    )(x, o, Wo, bo, ln2_g, ln2_b, Wr)


def _positions_body(topi_ref, rank_ref, c0_ref, c1_ref, pos_out, be_out):
    i = pl.program_id(0)
    E = NUM_EXPERTS
    counts = c0_ref[...] + c1_ref[...]
    padded = jnp.ceil(counts * (1.0 / M_BLK)) * M_BLK
    inc = padded
    s = 1
    while s < E:
        inc = inc + jnp.concatenate(
            [jnp.zeros((1, s), jnp.float32), inc[:, :-s]], 1)
        s *= 2
    start = inc - padded
    iota = lax.broadcasted_iota(jnp.int32, (S_BLK, E), 1)
    oh0 = (iota == topi_ref[:, 0:1]).astype(jnp.float32)
    oh1 = (iota == topi_ref[:, 1:2]).astype(jnp.float32)
    base0 = jnp.sum(oh0 * start, axis=1, keepdims=True)
    base1 = jnp.sum(oh1 * (start + c0_ref[...]), axis=1, keepdims=True)
    rank = rank_ref[...].astype(jnp.float32)
    pos0 = base0 + rank[:, 0:1]
    pos1 = base1 + rank[:, 1:2]
    pos_out[...] = jnp.concatenate([pos0, pos1], 1).astype(jnp.int32)

    @pl.when(i == 0)
    def _():
        ends = start + padded
        blk = lax.broadcasted_iota(
            jnp.int32, (1, NUM_M_BLKS), 1).astype(jnp.float32) * jnp.float32(M_BLK)
        acc = jnp.zeros((1, NUM_M_BLKS), jnp.float32)
        for e in range(E):
            acc += (blk >= ends[:, e:e + 1]).astype(jnp.float32)
        be_out[...] = jnp.minimum(acc, E - 1).astype(jnp.int32)


def _positions(topi, rank, counts0, counts1):
    E = NUM_EXPERTS
    nblk = N_TOK // S_BLK
    two = pl.BlockSpec((S_BLK, TOPK), lambda i: (i, 0))
    acc = pl.BlockSpec((1, E), lambda i: (0, 0))
    bes = pl.BlockSpec((1, NUM_M_BLKS), lambda i: (0, 0))
    return pl.pallas_call(
        _positions_body,
        grid=(nblk,),
        in_specs=[two, two, acc, acc],
        out_specs=[two, bes],
        out_shape=[
            jax.ShapeDtypeStruct((N_TOK, TOPK), jnp.int32),
            jax.ShapeDtypeStruct((1, NUM_M_BLKS), jnp.int32),
        ],
    )(topi, rank, counts0, counts1)


def _moe_body(be_ref, xs_ref, w1_ref, b1_ref, w2_ref, b2_ref, out_ref):
    h = jnp.dot(xs_ref[...], w1_ref[0], preferred_element_type=jnp.float32)
    h = h + b1_ref[0]
    h = 0.5 * h * (1.0 + lax.erf(h * jnp.float32(0.7071067811865476)))
    y = jnp.dot(h, w2_ref[0], preferred_element_type=jnp.float32)
    out_ref[...] = y + b2_ref[0]


def _moe_grouped(block_expert, xs, W1, b1, W2, b2):
    D, F = EMBED_DIM, FF
    grid_spec = pltpu.PrefetchScalarGridSpec(
        num_scalar_prefetch=1,
        grid=(NUM_M_BLKS,),
        in_specs=[
            pl.BlockSpec((M_BLK, D), lambda i, be: (i, 0)),
            pl.BlockSpec((1, D, F), lambda i, be: (be[i], 0, 0)),
            pl.BlockSpec((1, 1, F), lambda i, be: (be[i], 0, 0)),
            pl.BlockSpec((1, F, D), lambda i, be: (be[i], 0, 0)),
            pl.BlockSpec((1, 1, D), lambda i, be: (be[i], 0, 0)),
        ],
        out_specs=pl.BlockSpec((M_BLK, D), lambda i, be: (i, 0)),
    )
    return pl.pallas_call(
        _moe_body,
        grid_spec=grid_spec,
        out_shape=jax.ShapeDtypeStruct((R_PAD, D), jnp.float32),
    )(block_expert, xs, W1, b1.reshape(NUM_EXPERTS, 1, F),
      W2, b2.reshape(NUM_EXPERTS, 1, D))


def _final_body(x2_ref, r0_ref, r1_ref, gates_ref, c0_ref, c1_ref, psum_ref,
                out_ref, lb_ref):
    i = pl.program_id(0)
    out_ref[...] = (x2_ref[...] + gates_ref[:, 0:1] * r0_ref[...]
                    + gates_ref[:, 1:2] * r1_ref[...])

    @pl.when(i == 0)
    def _():
        counts = c0_ref[...] + c1_ref[...]
        frac = counts / jnp.float32(N_TOK * TOPK)
        pmean = psum_ref[...] / jnp.float32(N_TOK)
        lb_ref[...] = (LB_W * NUM_EXPERTS) * jnp.sum(
            frac * pmean, keepdims=True).reshape(1, 1)


def _final(x2, rows, gates, counts0, counts1, psum):
    D, E = EMBED_DIM, NUM_EXPERTS
    nblk = N_TOK // S_BLK
    row = pl.BlockSpec((S_BLK, D), lambda i: (i, 0))
    row1 = pl.BlockSpec((S_BLK, D), lambda i: (i + nblk, 0))
    two = pl.BlockSpec((S_BLK, TOPK), lambda i: (i, 0))
    acc = pl.BlockSpec((1, E), lambda i: (0, 0))
    one = pl.BlockSpec((1, 1), lambda i: (0, 0))
    return pl.pallas_call(
        _final_body,
        grid=(nblk,),
        in_specs=[row, row, row1, two, acc, acc, acc],
        out_specs=[row, one],
        out_shape=[
            jax.ShapeDtypeStruct((N_TOK, D), jnp.float32),
            jax.ShapeDtypeStruct((1, 1), jnp.float32),
        ],
    )(x2, rows, rows, gates, counts0, counts1, psum)


def kernel(x, ln1_g, ln1_b, ln2_g, ln2_b, Wq, bq, Wk, bk, Wv, bv,
           Wo, bo, Wr, W1, b1, W2, b2):
    B, S, D = x.shape
    x2d = x.reshape(S, D)
    v1 = lambda a: a.reshape(1, D)
    q, k, v = _qkv(x2d, v1(ln1_g), v1(ln1_b), Wq, v1(bq), Wk, v1(bk), Wv, v1(bv))
    H, dh = NUM_HEADS, D // NUM_HEADS
    to3 = lambda a: a.reshape(S, H, dh).transpose(1, 0, 2)
    o3 = _attention(to3(q), to3(k), to3(v))
    o = o3.transpose(1, 0, 2).reshape(S, D)
    x2, t, topi, gates, counts0, counts1, psum, rank = _router(
        x2d, o, Wo, v1(bo), v1(ln2_g), v1(ln2_b), Wr)
    pos, be = _positions(topi, rank, counts0, counts1)
    poscat = jnp.concatenate([pos[:, 0], pos[:, 1]])
    xs = _sc_dispatch_scatter(t, poscat)
    ys = _moe_grouped(be.reshape(NUM_M_BLKS), xs, W1, b1, W2, b2)
    rows = _sc_gather_rows(ys, poscat, N_ENTRY, D)
    out, lb = _final(x2, rows, gates, counts0, counts1, psum)
    return (out.reshape(B, S, D), lb.reshape(()))


# speedup vs baseline: 2.8324x; 1.0009x over previous
import functools

import jax
import jax.numpy as jnp
from jax import lax
from jax.experimental import pallas as pl
from jax.experimental.pallas import tpu as pltpu
from jax.experimental.pallas import tpu_sc as plsc

EMBED_DIM = 768
NUM_HEADS = 12
NUM_EXPERTS = 8
TOPK = 2
LB_W = 0.01
FF = EMBED_DIM * 4

S_BLK = 256
Q_BLK = 1024
M_BLK = 128
F_BLK = 768
N_TOK = 2048
N_ENTRY = N_TOK * TOPK
R_PAD = N_ENTRY + NUM_EXPERTS * M_BLK
NUM_M_BLKS = R_PAD // M_BLK
NUM_F_BLKS = FF // F_BLK

_SC_NC = 2
_SC_NS = 16
_SC_NW = _SC_NC * _SC_NS


def _sc_gather_rows(table, idx, nrows, ncols):
    b_per_w = nrows // _SC_NW
    mesh = plsc.VectorSubcoreMesh(core_axis_name="c", subcore_axis_name="s")

    @functools.partial(
        pl.kernel, mesh=mesh,
        out_type=jax.ShapeDtypeStruct((nrows, ncols), jnp.float32),
        compiler_params=pltpu.CompilerParams(use_tc_tiling_on_sc=True),
        scratch_types=[
            pltpu.VMEM((b_per_w,), jnp.int32),
            pltpu.VMEM((b_per_w, ncols), jnp.float32),
            pltpu.SemaphoreType.DMA,
        ],
    )
    def k(table_hbm, idx_hbm, out_hbm, idx_v, rows_v, sem):
        wid = lax.axis_index("s") * _SC_NC + lax.axis_index("c")
        base = wid * b_per_w
        pltpu.sync_copy(idx_hbm.at[pl.ds(base, b_per_w)], idx_v)
        pltpu.async_copy(table_hbm.at[idx_v], rows_v, sem).wait()
        pltpu.sync_copy(rows_v, out_hbm.at[pl.ds(base, b_per_w)])

    return k(table, idx)


def _sc_dispatch_scatter(t, poscat):
    b_per_w = N_ENTRY // _SC_NW
    mesh = plsc.VectorSubcoreMesh(core_axis_name="c", subcore_axis_name="s")

    @functools.partial(
        pl.kernel, mesh=mesh,
        out_type=jax.ShapeDtypeStruct((R_PAD, EMBED_DIM), jnp.float32),
        compiler_params=pltpu.CompilerParams(use_tc_tiling_on_sc=True),
        scratch_types=[
            pltpu.VMEM((b_per_w,), jnp.int32),
            pltpu.VMEM((b_per_w, EMBED_DIM), jnp.float32),
            pltpu.SemaphoreType.DMA,
        ],
    )
    def k(t_hbm, pos_hbm, xs_hbm, pos_v, rows_v, sem):
        wid = lax.axis_index("s") * _SC_NC + lax.axis_index("c")
        base = wid * b_per_w
        tok0 = base - (base // N_TOK) * N_TOK
        pltpu.sync_copy(pos_hbm.at[pl.ds(base, b_per_w)], pos_v)
        pltpu.sync_copy(t_hbm.at[pl.ds(tok0, b_per_w)], rows_v)
        pltpu.async_copy(rows_v, xs_hbm.at[pos_v], sem).wait()

    return k(t, poscat)


def _ln(x, g, b):
    m = jnp.mean(x, -1, keepdims=True)
    v = jnp.mean((x - m) * (x - m), -1, keepdims=True)
    return (x - m) * lax.rsqrt(v + 1e-5) * g + b


def _qkv_body(x_ref, g_ref, b_ref, wq_ref, bq_ref, wk_ref, bk_ref,
              wv_ref, bv_ref, q_out, k_out, v_out):
    h = _ln(x_ref[...], g_ref[...], b_ref[...])
    q_out[...] = jnp.dot(h, wq_ref[...], preferred_element_type=jnp.float32) + bq_ref[...]
    k_out[...] = jnp.dot(h, wk_ref[...], preferred_element_type=jnp.float32) + bk_ref[...]
    v_out[...] = jnp.dot(h, wv_ref[...], preferred_element_type=jnp.float32) + bv_ref[...]


def _qkv(x, ln1_g, ln1_b, Wq, bq, Wk, bk, Wv, bv):
    D = EMBED_DIM
    nblk = N_TOK // S_BLK
    row = pl.BlockSpec((S_BLK, D), lambda i: (i, 0))
    full = pl.BlockSpec((D, D), lambda i: (0, 0))
    vec = pl.BlockSpec((1, D), lambda i: (0, 0))
    out = jax.ShapeDtypeStruct((N_TOK, D), jnp.float32)
    return pl.pallas_call(
        _qkv_body,
        grid=(nblk,),
        in_specs=[row, vec, vec, full, vec, full, vec, full, vec],
        out_specs=[row, row, row],
        out_shape=[out, out, out],
        compiler_params=pltpu.CompilerParams(
            dimension_semantics=("parallel",)),
    )(x, ln1_g, ln1_b, Wq, bq, Wk, bk, Wv, bv)


def _attn_body(q_ref, k_ref, v_ref, o_ref):
    dh = EMBED_DIM // NUM_HEADS
    s = lax.dot_general(q_ref[0], k_ref[0],
                        (((1,), (1,)), ((), ())),
                        preferred_element_type=jnp.float32)
    s = s * (1.0 / jnp.sqrt(jnp.float32(dh)))
    m = jnp.max(s, axis=-1, keepdims=True)
    p = jnp.exp(s - m)
    o = jnp.dot(p, v_ref[0], preferred_element_type=jnp.float32)
    o_ref[0] = o * (1.0 / jnp.sum(p, axis=-1, keepdims=True))


def _attention(q3, k3, v3):
    dh = EMBED_DIM // NUM_HEADS
    nq = N_TOK // Q_BLK
    qspec = pl.BlockSpec((1, Q_BLK, dh), lambda h, i: (h, i, 0))
    kvspec = pl.BlockSpec((1, N_TOK, dh), lambda h, i: (h, 0, 0))
    return pl.pallas_call(
        _attn_body,
        grid=(NUM_HEADS, nq),
        in_specs=[qspec, kvspec, kvspec],
        out_specs=qspec,
        out_shape=jax.ShapeDtypeStruct((NUM_HEADS, N_TOK, dh), jnp.float32),
        compiler_params=pltpu.CompilerParams(
            dimension_semantics=("parallel", "parallel")),
    )(q3, k3, v3)


def _cumsum_rows(x):
    n = x.shape[0]
    s = 1
    while s < n:
        x = x + jnp.concatenate([jnp.zeros((s, x.shape[1]), x.dtype), x[:-s]], 0)
        s *= 2
    return x


def _router_body(x_ref, o_ref, wo_ref, bo_ref, g_ref, b_ref, wr_ref,
                 x2_out, t_out, topi_out, gates_out,
                 counts0_out, counts1_out, psum_out, rank_out):
    i = pl.program_id(0)
    E = NUM_EXPERTS
    x2 = x_ref[...] + jnp.dot(o_ref[...], wo_ref[...],
                              preferred_element_type=jnp.float32) + bo_ref[...]
    x2_out[...] = x2
    t = _ln(x2, g_ref[...], b_ref[...])
    t_out[...] = t
    logits = jnp.dot(t, wr_ref[...], preferred_element_type=jnp.float32)
    lm = jnp.max(logits, axis=-1, keepdims=True)
    pe = jnp.exp(logits - lm)
    probs = pe / jnp.sum(pe, axis=-1, keepdims=True)
    iota = lax.broadcasted_iota(jnp.int32, probs.shape, 1)
    m1 = jnp.max(probs, axis=-1, keepdims=True)
    i1 = jnp.min(jnp.where(probs == m1, iota, E), axis=-1, keepdims=True)
    probs2 = jnp.where(iota == i1, -1.0, probs)
    m2 = jnp.max(probs2, axis=-1, keepdims=True)
    i2 = jnp.min(jnp.where(probs2 == m2, iota, E), axis=-1, keepdims=True)
    denom = m1 + m2
    topi_out[...] = jnp.concatenate([i1, i2], axis=1)
    gates_out[...] = jnp.concatenate([m1 / denom, m2 / denom], axis=1)
    onehot0 = (iota == i1).astype(jnp.float32)
    onehot1 = (iota == i2).astype(jnp.float32)
    ps = jnp.sum(probs, axis=0, keepdims=True)

    @pl.when(i == 0)
    def _():
        counts0_out[...] = jnp.zeros_like(counts0_out)
        counts1_out[...] = jnp.zeros_like(counts1_out)
        psum_out[...] = jnp.zeros_like(psum_out)

    off0 = counts0_out[...]
    off1 = counts1_out[...]
    exc0 = _cumsum_rows(onehot0) - onehot0
    exc1 = _cumsum_rows(onehot1) - onehot1
    rank0 = jnp.sum(onehot0 * (off0 + exc0), axis=1, keepdims=True)
    rank1 = jnp.sum(onehot1 * (off1 + exc1), axis=1, keepdims=True)
    rank_out[...] = jnp.concatenate([rank0, rank1], 1).astype(jnp.int32)
    counts0_out[...] = off0 + jnp.sum(onehot0, axis=0, keepdims=True)
    counts1_out[...] = off1 + jnp.sum(onehot1, axis=0, keepdims=True)
    psum_out[...] += ps


def _router(x, o, Wo, bo, ln2_g, ln2_b, Wr):
    D, E = EMBED_DIM, NUM_EXPERTS
    nblk = N_TOK // S_BLK
    row = pl.BlockSpec((S_BLK, D), lambda i: (i, 0))
    full = pl.BlockSpec((D, D), lambda i: (0, 0))
    vec = pl.BlockSpec((1, D), lambda i: (0, 0))
    wr = pl.BlockSpec((D, E), lambda i: (0, 0))
    two = pl.BlockSpec((S_BLK, TOPK), lambda i: (i, 0))
    acc = pl.BlockSpec((1, E), lambda i: (0, 0))
    return pl.pallas_call(
        _router_body,
        grid=(nblk,),
        in_specs=[row, row, full, vec, vec, vec, wr],
        out_specs=[row, row, two, two, acc, acc, acc, two],
        out_shape=[
            jax.ShapeDtypeStruct((N_TOK, D), jnp.float32),
            jax.ShapeDtypeStruct((N_TOK, D), jnp.float32),
            jax.ShapeDtypeStruct((N_TOK, TOPK), jnp.int32),
            jax.ShapeDtypeStruct((N_TOK, TOPK), jnp.float32),
            jax.ShapeDtypeStruct((1, E), jnp.float32),
            jax.ShapeDtypeStruct((1, E), jnp.float32),
            jax.ShapeDtypeStruct((1, E), jnp.float32),
            jax.ShapeDtypeStruct((N_TOK, TOPK), jnp.int32),
        ],
    )(x, o, Wo, bo, ln2_g, ln2_b, Wr)


def _positions_body(topi_ref, rank_ref, c0_ref, c1_ref, pos_out, be_out):
    i = pl.program_id(0)
    E = NUM_EXPERTS
    counts = c0_ref[...] + c1_ref[...]
    padded = jnp.ceil(counts * (1.0 / M_BLK)) * M_BLK
    inc = padded
    s = 1
    while s < E:
        inc = inc + jnp.concatenate(
            [jnp.zeros((1, s), jnp.float32), inc[:, :-s]], 1)
        s *= 2
    start = inc - padded
    iota = lax.broadcasted_iota(jnp.int32, (S_BLK, E), 1)
    oh0 = (iota == topi_ref[:, 0:1]).astype(jnp.float32)
    oh1 = (iota == topi_ref[:, 1:2]).astype(jnp.float32)
    base0 = jnp.sum(oh0 * start, axis=1, keepdims=True)
    base1 = jnp.sum(oh1 * (start + c0_ref[...]), axis=1, keepdims=True)
    rank = rank_ref[...].astype(jnp.float32)
    pos0 = base0 + rank[:, 0:1]
    pos1 = base1 + rank[:, 1:2]
    pos_out[...] = jnp.concatenate([pos0, pos1], 1).astype(jnp.int32)

    @pl.when(i == 0)
    def _():
        ends = start + padded
        blk = lax.broadcasted_iota(
            jnp.int32, (1, NUM_M_BLKS), 1).astype(jnp.float32) * jnp.float32(M_BLK)
        acc = jnp.zeros((1, NUM_M_BLKS), jnp.float32)
        for e in range(E):
            acc += (blk >= ends[:, e:e + 1]).astype(jnp.float32)
        be_out[...] = jnp.minimum(acc, E - 1).astype(jnp.int32)


def _positions(topi, rank, counts0, counts1):
    E = NUM_EXPERTS
    nblk = N_TOK // S_BLK
    two = pl.BlockSpec((S_BLK, TOPK), lambda i: (i, 0))
    acc = pl.BlockSpec((1, E), lambda i: (0, 0))
    bes = pl.BlockSpec((1, NUM_M_BLKS), lambda i: (0, 0))
    return pl.pallas_call(
        _positions_body,
        grid=(nblk,),
        in_specs=[two, two, acc, acc],
        out_specs=[two, bes],
        out_shape=[
            jax.ShapeDtypeStruct((N_TOK, TOPK), jnp.int32),
            jax.ShapeDtypeStruct((1, NUM_M_BLKS), jnp.int32),
        ],
        compiler_params=pltpu.CompilerParams(
            dimension_semantics=("parallel",)),
    )(topi, rank, counts0, counts1)


def _moe_body(be_ref, xs_ref, w1_ref, b1_ref, w2_ref, b2_ref, out_ref):
    h = jnp.dot(xs_ref[...], w1_ref[0], preferred_element_type=jnp.float32)
    h = h + b1_ref[0]
    h = 0.5 * h * (1.0 + lax.erf(h * jnp.float32(0.7071067811865476)))
    y = jnp.dot(h, w2_ref[0], preferred_element_type=jnp.float32)
    out_ref[...] = y + b2_ref[0]


def _moe_grouped(block_expert, xs, W1, b1, W2, b2):
    D, F = EMBED_DIM, FF
    grid_spec = pltpu.PrefetchScalarGridSpec(
        num_scalar_prefetch=1,
        grid=(NUM_M_BLKS,),
        in_specs=[
            pl.BlockSpec((M_BLK, D), lambda i, be: (i, 0)),
            pl.BlockSpec((1, D, F), lambda i, be: (be[i], 0, 0)),
            pl.BlockSpec((1, 1, F), lambda i, be: (be[i], 0, 0)),
            pl.BlockSpec((1, F, D), lambda i, be: (be[i], 0, 0)),
            pl.BlockSpec((1, 1, D), lambda i, be: (be[i], 0, 0)),
        ],
        out_specs=pl.BlockSpec((M_BLK, D), lambda i, be: (i, 0)),
    )
    return pl.pallas_call(
        _moe_body,
        grid_spec=grid_spec,
        out_shape=jax.ShapeDtypeStruct((R_PAD, D), jnp.float32),
        compiler_params=pltpu.CompilerParams(
            dimension_semantics=("parallel",)),
    )(block_expert, xs, W1, b1.reshape(NUM_EXPERTS, 1, F),
      W2, b2.reshape(NUM_EXPERTS, 1, D))


def _final_body(x2_ref, r0_ref, r1_ref, gates_ref, c0_ref, c1_ref, psum_ref,
                out_ref, lb_ref):
    i = pl.program_id(0)
    out_ref[...] = (x2_ref[...] + gates_ref[:, 0:1] * r0_ref[...]
                    + gates_ref[:, 1:2] * r1_ref[...])

    @pl.when(i == 0)
    def _():
        counts = c0_ref[...] + c1_ref[...]
        frac = counts / jnp.float32(N_TOK * TOPK)
        pmean = psum_ref[...] / jnp.float32(N_TOK)
        lb_ref[...] = (LB_W * NUM_EXPERTS) * jnp.sum(
            frac * pmean, keepdims=True).reshape(1, 1)


def _final(x2, rows, gates, counts0, counts1, psum):
    D, E = EMBED_DIM, NUM_EXPERTS
    nblk = N_TOK // S_BLK
    row = pl.BlockSpec((S_BLK, D), lambda i: (i, 0))
    row1 = pl.BlockSpec((S_BLK, D), lambda i: (i + nblk, 0))
    two = pl.BlockSpec((S_BLK, TOPK), lambda i: (i, 0))
    acc = pl.BlockSpec((1, E), lambda i: (0, 0))
    one = pl.BlockSpec((1, 1), lambda i: (0, 0))
    return pl.pallas_call(
        _final_body,
        grid=(nblk,),
        in_specs=[row, row, row1, two, acc, acc, acc],
        out_specs=[row, one],
        out_shape=[
            jax.ShapeDtypeStruct((N_TOK, D), jnp.float32),
            jax.ShapeDtypeStruct((1, 1), jnp.float32),
        ],
        compiler_params=pltpu.CompilerParams(
            dimension_semantics=("parallel",)),
    )(x2, rows, rows, gates, counts0, counts1, psum)


def kernel(x, ln1_g, ln1_b, ln2_g, ln2_b, Wq, bq, Wk, bk, Wv, bv,
           Wo, bo, Wr, W1, b1, W2, b2):
    B, S, D = x.shape
    x2d = x.reshape(S, D)
    v1 = lambda a: a.reshape(1, D)
    q, k, v = _qkv(x2d, v1(ln1_g), v1(ln1_b), Wq, v1(bq), Wk, v1(bk), Wv, v1(bv))
    H, dh = NUM_HEADS, D // NUM_HEADS
    to3 = lambda a: a.reshape(S, H, dh).transpose(1, 0, 2)
    o3 = _attention(to3(q), to3(k), to3(v))
    o = o3.transpose(1, 0, 2).reshape(S, D)
    x2, t, topi, gates, counts0, counts1, psum, rank = _router(
        x2d, o, Wo, v1(bo), v1(ln2_g), v1(ln2_b), Wr)
    pos, be = _positions(topi, rank, counts0, counts1)
    poscat = jnp.concatenate([pos[:, 0], pos[:, 1]])
    xs = _sc_dispatch_scatter(t, poscat)
    ys = _moe_grouped(be.reshape(NUM_M_BLKS), xs, W1, b1, W2, b2)
    rows = _sc_gather_rows(ys, poscat, N_ENTRY, D)
    out, lb = _final(x2, rows, gates, counts0, counts1, psum)
    return (out.reshape(B, S, D), lb.reshape(()))


# speedup vs baseline: 3.5120x; 1.2399x over previous
import functools

import jax
import jax.numpy as jnp
from jax import lax
from jax.experimental import pallas as pl
from jax.experimental.pallas import tpu as pltpu
from jax.experimental.pallas import tpu_sc as plsc

EMBED_DIM = 768
NUM_HEADS = 12
NUM_EXPERTS = 8
TOPK = 2
LB_W = 0.01
FF = EMBED_DIM * 4

S_BLK = 256
Q_BLK = 1024
M_BLK = 128
F_BLK = 768
N_TOK = 2048
N_ENTRY = N_TOK * TOPK
R_PAD = N_ENTRY + NUM_EXPERTS * M_BLK
NUM_M_BLKS = R_PAD // M_BLK
NUM_F_BLKS = FF // F_BLK

_SC_NC = 2
_SC_NS = 16
_SC_NW = _SC_NC * _SC_NS


def _sc_gather_rows(table, idx, nrows, ncols):
    b_per_w = nrows // _SC_NW
    mesh = plsc.VectorSubcoreMesh(core_axis_name="c", subcore_axis_name="s")

    @functools.partial(
        pl.kernel, mesh=mesh,
        out_type=jax.ShapeDtypeStruct((nrows, ncols), jnp.float32),
        compiler_params=pltpu.CompilerParams(use_tc_tiling_on_sc=True),
        scratch_types=[
            pltpu.VMEM((b_per_w,), jnp.int32),
            pltpu.VMEM((b_per_w, ncols), jnp.float32),
            pltpu.SemaphoreType.DMA,
        ],
    )
    def k(table_hbm, idx_hbm, out_hbm, idx_v, rows_v, sem):
        wid = lax.axis_index("s") * _SC_NC + lax.axis_index("c")
        base = wid * b_per_w
        pltpu.sync_copy(idx_hbm.at[pl.ds(base, b_per_w)], idx_v)
        pltpu.async_copy(table_hbm.at[idx_v], rows_v, sem).wait()
        pltpu.sync_copy(rows_v, out_hbm.at[pl.ds(base, b_per_w)])

    return k(table, idx)


def _sc_dispatch_scatter(t, poscat):
    b_per_w = N_ENTRY // _SC_NW
    mesh = plsc.VectorSubcoreMesh(core_axis_name="c", subcore_axis_name="s")

    @functools.partial(
        pl.kernel, mesh=mesh,
        out_type=jax.ShapeDtypeStruct((R_PAD, EMBED_DIM), jnp.float32),
        compiler_params=pltpu.CompilerParams(use_tc_tiling_on_sc=True),
        scratch_types=[
            pltpu.VMEM((b_per_w,), jnp.int32),
            pltpu.VMEM((b_per_w, EMBED_DIM), jnp.float32),
            pltpu.SemaphoreType.DMA,
        ],
    )
    def k(t_hbm, pos_hbm, xs_hbm, pos_v, rows_v, sem):
        wid = lax.axis_index("s") * _SC_NC + lax.axis_index("c")
        base = wid * b_per_w
        tok0 = base - (base // N_TOK) * N_TOK
        pltpu.sync_copy(pos_hbm.at[pl.ds(base, b_per_w)], pos_v)
        pltpu.sync_copy(t_hbm.at[pl.ds(tok0, b_per_w)], rows_v)
        pltpu.async_copy(rows_v, xs_hbm.at[pos_v], sem).wait()

    return k(t, poscat)


def _ln(x, g, b):
    m = jnp.mean(x, -1, keepdims=True)
    v = jnp.mean((x - m) * (x - m), -1, keepdims=True)
    return (x - m) * lax.rsqrt(v + 1e-5) * g + b


def _qkv_body(x_ref, g_ref, b_ref, wq_ref, bq_ref, wk_ref, bk_ref,
              wv_ref, bv_ref, q_out, k_out, v_out):
    h = _ln(x_ref[...], g_ref[...], b_ref[...])
    q_out[...] = jnp.dot(h, wq_ref[...], preferred_element_type=jnp.float32) + bq_ref[...]
    k_out[...] = jnp.dot(h, wk_ref[...], preferred_element_type=jnp.float32) + bk_ref[...]
    v_out[...] = jnp.dot(h, wv_ref[...], preferred_element_type=jnp.float32) + bv_ref[...]


def _qkv(x, ln1_g, ln1_b, Wq, bq, Wk, bk, Wv, bv):
    D = EMBED_DIM
    nblk = N_TOK // S_BLK
    row = pl.BlockSpec((S_BLK, D), lambda i: (i, 0))
    full = pl.BlockSpec((D, D), lambda i: (0, 0))
    vec = pl.BlockSpec((1, D), lambda i: (0, 0))
    out = jax.ShapeDtypeStruct((N_TOK, D), jnp.float32)
    return pl.pallas_call(
        _qkv_body,
        grid=(nblk,),
        in_specs=[row, vec, vec, full, vec, full, vec, full, vec],
        out_specs=[row, row, row],
        out_shape=[out, out, out],
        compiler_params=pltpu.CompilerParams(
            dimension_semantics=("parallel",)),
    )(x, ln1_g, ln1_b, Wq, bq, Wk, bk, Wv, bv)


def _attn_body(q_ref, k_ref, v_ref, o_ref):
    dh = EMBED_DIM // NUM_HEADS
    scale = 1.0 / jnp.sqrt(jnp.float32(dh))
    for sub in range(2):
        lo, hi = sub * dh, (sub + 1) * dh
        s = lax.dot_general(q_ref[:, lo:hi], k_ref[:, lo:hi],
                            (((1,), (1,)), ((), ())),
                            preferred_element_type=jnp.float32) * scale
        m = jnp.max(s, axis=-1, keepdims=True)
        p = jnp.exp(s - m)
        o = jnp.dot(p, v_ref[:, lo:hi], preferred_element_type=jnp.float32)
        o_ref[:, lo:hi] = o * (1.0 / jnp.sum(p, axis=-1, keepdims=True))


def _attention(q, k, v):
    dh2 = 2 * (EMBED_DIM // NUM_HEADS)
    nq = N_TOK // Q_BLK
    qspec = pl.BlockSpec((Q_BLK, dh2), lambda p, i: (i, p))
    kvspec = pl.BlockSpec((N_TOK, dh2), lambda p, i: (0, p))
    return pl.pallas_call(
        _attn_body,
        grid=(NUM_HEADS // 2, nq),
        in_specs=[qspec, kvspec, kvspec],
        out_specs=qspec,
        out_shape=jax.ShapeDtypeStruct((N_TOK, EMBED_DIM), jnp.float32),
        compiler_params=pltpu.CompilerParams(
            dimension_semantics=("parallel", "parallel")),
    )(q, k, v)


def _cumsum_rows(x):
    n = x.shape[0]
    s = 1
    while s < n:
        x = x + jnp.concatenate([jnp.zeros((s, x.shape[1]), x.dtype), x[:-s]], 0)
        s *= 2
    return x


def _router_body(x_ref, o_ref, wo_ref, bo_ref, g_ref, b_ref, wr_ref,
                 x2_out, t_out, topi_out, gates_out,
                 counts0_out, counts1_out, psum_out, rank_out):
    i = pl.program_id(0)
    E = NUM_EXPERTS
    x2 = x_ref[...] + jnp.dot(o_ref[...], wo_ref[...],
                              preferred_element_type=jnp.float32) + bo_ref[...]
    x2_out[...] = x2
    t = _ln(x2, g_ref[...], b_ref[...])
    t_out[...] = t
    logits = jnp.dot(t, wr_ref[...], preferred_element_type=jnp.float32)
    lm = jnp.max(logits, axis=-1, keepdims=True)
    pe = jnp.exp(logits - lm)
    probs = pe / jnp.sum(pe, axis=-1, keepdims=True)
    iota = lax.broadcasted_iota(jnp.int32, probs.shape, 1)
    m1 = jnp.max(probs, axis=-1, keepdims=True)
    i1 = jnp.min(jnp.where(probs == m1, iota, E), axis=-1, keepdims=True)
    probs2 = jnp.where(iota == i1, -1.0, probs)
    m2 = jnp.max(probs2, axis=-1, keepdims=True)
    i2 = jnp.min(jnp.where(probs2 == m2, iota, E), axis=-1, keepdims=True)
    denom = m1 + m2
    topi_out[...] = jnp.concatenate([i1, i2], axis=1)
    gates_out[...] = jnp.concatenate([m1 / denom, m2 / denom], axis=1)
    onehot0 = (iota == i1).astype(jnp.float32)
    onehot1 = (iota == i2).astype(jnp.float32)
    ps = jnp.sum(probs, axis=0, keepdims=True)

    @pl.when(i == 0)
    def _():
        counts0_out[...] = jnp.zeros_like(counts0_out)
        counts1_out[...] = jnp.zeros_like(counts1_out)
        psum_out[...] = jnp.zeros_like(psum_out)

    off0 = counts0_out[...]
    off1 = counts1_out[...]
    exc0 = _cumsum_rows(onehot0) - onehot0
    exc1 = _cumsum_rows(onehot1) - onehot1
    rank0 = jnp.sum(onehot0 * (off0 + exc0), axis=1, keepdims=True)
    rank1 = jnp.sum(onehot1 * (off1 + exc1), axis=1, keepdims=True)
    rank_out[...] = jnp.concatenate([rank0, rank1], 1).astype(jnp.int32)
    counts0_out[...] = off0 + jnp.sum(onehot0, axis=0, keepdims=True)
    counts1_out[...] = off1 + jnp.sum(onehot1, axis=0, keepdims=True)
    psum_out[...] += ps


def _router(x, o, Wo, bo, ln2_g, ln2_b, Wr):
    D, E = EMBED_DIM, NUM_EXPERTS
    nblk = N_TOK // S_BLK
    row = pl.BlockSpec((S_BLK, D), lambda i: (i, 0))
    full = pl.BlockSpec((D, D), lambda i: (0, 0))
    vec = pl.BlockSpec((1, D), lambda i: (0, 0))
    wr = pl.BlockSpec((D, E), lambda i: (0, 0))
    two = pl.BlockSpec((S_BLK, TOPK), lambda i: (i, 0))
    acc = pl.BlockSpec((1, E), lambda i: (0, 0))
    return pl.pallas_call(
        _router_body,
        grid=(nblk,),
        in_specs=[row, row, full, vec, vec, vec, wr],
        out_specs=[row, row, two, two, acc, acc, acc, two],
        out_shape=[
            jax.ShapeDtypeStruct((N_TOK, D), jnp.float32),
            jax.ShapeDtypeStruct((N_TOK, D), jnp.float32),
            jax.ShapeDtypeStruct((N_TOK, TOPK), jnp.int32),
            jax.ShapeDtypeStruct((N_TOK, TOPK), jnp.float32),
            jax.ShapeDtypeStruct((1, E), jnp.float32),
            jax.ShapeDtypeStruct((1, E), jnp.float32),
            jax.ShapeDtypeStruct((1, E), jnp.float32),
            jax.ShapeDtypeStruct((N_TOK, TOPK), jnp.int32),
        ],
    )(x, o, Wo, bo, ln2_g, ln2_b, Wr)


def _positions_body(topi_ref, rank_ref, c0_ref, c1_ref, pos_out, be_out):
    i = pl.program_id(0)
    E = NUM_EXPERTS
    counts = c0_ref[...] + c1_ref[...]
    padded = jnp.ceil(counts * (1.0 / M_BLK)) * M_BLK
    inc = padded
    s = 1
    while s < E:
        inc = inc + jnp.concatenate(
            [jnp.zeros((1, s), jnp.float32), inc[:, :-s]], 1)
        s *= 2
    start = inc - padded
    iota = lax.broadcasted_iota(jnp.int32, (S_BLK, E), 1)
    oh0 = (iota == topi_ref[:, 0:1]).astype(jnp.float32)
    oh1 = (iota == topi_ref[:, 1:2]).astype(jnp.float32)
    base0 = jnp.sum(oh0 * start, axis=1, keepdims=True)
    base1 = jnp.sum(oh1 * (start + c0_ref[...]), axis=1, keepdims=True)
    rank = rank_ref[...].astype(jnp.float32)
    pos0 = base0 + rank[:, 0:1]
    pos1 = base1 + rank[:, 1:2]
    pos_out[...] = jnp.concatenate([pos0, pos1], 1).astype(jnp.int32)

    @pl.when(i == 0)
    def _():
        ends = start + padded
        blk = lax.broadcasted_iota(
            jnp.int32, (1, NUM_M_BLKS), 1).astype(jnp.float32) * jnp.float32(M_BLK)
        acc = jnp.zeros((1, NUM_M_BLKS), jnp.float32)
        for e in range(E):
            acc += (blk >= ends[:, e:e + 1]).astype(jnp.float32)
        be_out[...] = jnp.minimum(acc, E - 1).astype(jnp.int32)


def _positions(topi, rank, counts0, counts1):
    E = NUM_EXPERTS
    nblk = N_TOK // S_BLK
    two = pl.BlockSpec((S_BLK, TOPK), lambda i: (i, 0))
    acc = pl.BlockSpec((1, E), lambda i: (0, 0))
    bes = pl.BlockSpec((1, NUM_M_BLKS), lambda i: (0, 0))
    return pl.pallas_call(
        _positions_body,
        grid=(nblk,),
        in_specs=[two, two, acc, acc],
        out_specs=[two, bes],
        out_shape=[
            jax.ShapeDtypeStruct((N_TOK, TOPK), jnp.int32),
            jax.ShapeDtypeStruct((1, NUM_M_BLKS), jnp.int32),
        ],
        compiler_params=pltpu.CompilerParams(
            dimension_semantics=("parallel",)),
    )(topi, rank, counts0, counts1)


def _moe_body(be_ref, xs_ref, w1_ref, b1_ref, w2_ref, b2_ref, out_ref):
    h = jnp.dot(xs_ref[...], w1_ref[0], preferred_element_type=jnp.float32)
    h = h + b1_ref[0]
    h = 0.5 * h * (1.0 + lax.erf(h * jnp.float32(0.7071067811865476)))
    y = jnp.dot(h, w2_ref[0], preferred_element_type=jnp.float32)
    out_ref[...] = y + b2_ref[0]


def _moe_grouped(block_expert, xs, W1, b1, W2, b2):
    D, F = EMBED_DIM, FF
    grid_spec = pltpu.PrefetchScalarGridSpec(
        num_scalar_prefetch=1,
        grid=(NUM_M_BLKS,),
        in_specs=[
            pl.BlockSpec((M_BLK, D), lambda i, be: (i, 0)),
            pl.BlockSpec((1, D, F), lambda i, be: (be[i], 0, 0)),
            pl.BlockSpec((1, 1, F), lambda i, be: (be[i], 0, 0)),
            pl.BlockSpec((1, F, D), lambda i, be: (be[i], 0, 0)),
            pl.BlockSpec((1, 1, D), lambda i, be: (be[i], 0, 0)),
        ],
        out_specs=pl.BlockSpec((M_BLK, D), lambda i, be: (i, 0)),
    )
    return pl.pallas_call(
        _moe_body,
        grid_spec=grid_spec,
        out_shape=jax.ShapeDtypeStruct((R_PAD, D), jnp.float32),
        compiler_params=pltpu.CompilerParams(
            dimension_semantics=("parallel",)),
    )(block_expert, xs, W1, b1.reshape(NUM_EXPERTS, 1, F),
      W2, b2.reshape(NUM_EXPERTS, 1, D))


def _final_body(x2_ref, r0_ref, r1_ref, gates_ref, c0_ref, c1_ref, psum_ref,
                out_ref, lb_ref):
    i = pl.program_id(0)
    out_ref[...] = (x2_ref[...] + gates_ref[:, 0:1] * r0_ref[...]
                    + gates_ref[:, 1:2] * r1_ref[...])

    @pl.when(i == 0)
    def _():
        counts = c0_ref[...] + c1_ref[...]
        frac = counts / jnp.float32(N_TOK * TOPK)
        pmean = psum_ref[...] / jnp.float32(N_TOK)
        lb_ref[...] = (LB_W * NUM_EXPERTS) * jnp.sum(
            frac * pmean, keepdims=True).reshape(1, 1)


def _final(x2, rows, gates, counts0, counts1, psum):
    D, E = EMBED_DIM, NUM_EXPERTS
    nblk = N_TOK // S_BLK
    row = pl.BlockSpec((S_BLK, D), lambda i: (i, 0))
    row1 = pl.BlockSpec((S_BLK, D), lambda i: (i + nblk, 0))
    two = pl.BlockSpec((S_BLK, TOPK), lambda i: (i, 0))
    acc = pl.BlockSpec((1, E), lambda i: (0, 0))
    one = pl.BlockSpec((1, 1), lambda i: (0, 0))
    return pl.pallas_call(
        _final_body,
        grid=(nblk,),
        in_specs=[row, row, row1, two, acc, acc, acc],
        out_specs=[row, one],
        out_shape=[
            jax.ShapeDtypeStruct((N_TOK, D), jnp.float32),
            jax.ShapeDtypeStruct((1, 1), jnp.float32),
        ],
        compiler_params=pltpu.CompilerParams(
            dimension_semantics=("parallel",)),
    )(x2, rows, rows, gates, counts0, counts1, psum)


def kernel(x, ln1_g, ln1_b, ln2_g, ln2_b, Wq, bq, Wk, bk, Wv, bv,
           Wo, bo, Wr, W1, b1, W2, b2):
    B, S, D = x.shape
    x2d = x.reshape(S, D)
    v1 = lambda a: a.reshape(1, D)
    q, k, v = _qkv(x2d, v1(ln1_g), v1(ln1_b), Wq, v1(bq), Wk, v1(bk), Wv, v1(bv))
    o = _attention(q, k, v)
    x2, t, topi, gates, counts0, counts1, psum, rank = _router(
        x2d, o, Wo, v1(bo), v1(ln2_g), v1(ln2_b), Wr)
    pos, be = _positions(topi, rank, counts0, counts1)
    poscat = jnp.concatenate([pos[:, 0], pos[:, 1]])
    xs = _sc_dispatch_scatter(t, poscat)
    ys = _moe_grouped(be.reshape(NUM_M_BLKS), xs, W1, b1, W2, b2)
    rows = _sc_gather_rows(ys, poscat, N_ENTRY, D)
    out, lb = _final(x2, rows, gates, counts0, counts1, psum)
    return (out.reshape(B, S, D), lb.reshape(()))
